# Initial kernel scaffold; baseline (speedup 1.0000x reference)
import jax
import jax.numpy as jnp
from jax.experimental import pallas as pl


def kernel(x, p, params):
    raise NotImplementedError("write your pallas kernel here")


# speedup vs baseline: 9.9962x; 9.9962x over previous
import functools

import jax
import jax.numpy as jnp
from jax import lax
from jax.experimental import pallas as pl
from jax.experimental.pallas import tpu as pltpu
from jax.experimental.pallas import tpu_sc as plsc


_SC_INFO = None


def _sc_info():
    global _SC_INFO
    if _SC_INFO is None:
        _SC_INFO = plsc.get_sparse_core_info()
    return _SC_INFO


@functools.lru_cache(maxsize=None)
def _gather_fn(R, D, Bn):
    info = _sc_info()
    NC, NS = info.num_cores, info.num_subcores
    NW = NC * NS
    bpw = Bn // NW
    c = 8
    for cand in range(min(128, bpw), 7, -8):
        if bpw % cand == 0 and cand * (D + 1) * 4 <= 480_000:
            c = cand
            break
    nch = bpw // c
    mesh = plsc.VectorSubcoreMesh(core_axis_name="c", subcore_axis_name="s")

    @functools.partial(
        pl.kernel,
        mesh=mesh,
        compiler_params=pltpu.CompilerParams(use_tc_tiling_on_sc=False),
        out_type=jax.ShapeDtypeStruct((Bn, D), jnp.float32),
        scratch_types=[
            pltpu.VMEM((c,), jnp.int32),
            pltpu.VMEM((c, D), jnp.float32),
            pltpu.SemaphoreType.DMA,
        ],
    )
    def k(table_hbm, idx_hbm, out_hbm, idx_v, rows_v, sem):
        wid = lax.axis_index("s") * NC + lax.axis_index("c")
        base = wid * bpw

        def step(j, carry):
            off = base + j * c
            pltpu.sync_copy(idx_hbm.at[pl.ds(off, c)], idx_v)
            pltpu.async_copy(table_hbm.at[idx_v], rows_v, sem).wait()
            pltpu.sync_copy(rows_v, out_hbm.at[pl.ds(off, c)])
            return carry

        lax.fori_loop(0, nch, step, 0)

    return k


def _gather_rows(table, idx):
    Bn0 = idx.shape[0]
    Bn = -(-Bn0 // 256) * 256
    if Bn != Bn0:
        idx = jnp.concatenate([idx, jnp.zeros((Bn - Bn0,), jnp.int32)])
    out = _gather_fn(table.shape[0], table.shape[1], Bn)(table, idx)
    return out[:Bn0]


@functools.lru_cache(maxsize=None)
def _knn_fn(B, Nq, Nk, K, Qb):
    def body(pq_ref, pkT_ref, idx_ref, dst_ref):
        b = pl.program_id(0)
        pq = pq_ref[0]
        pkT = pkT_ref[0]
        qx, qy, qz = pq[:, 0:1], pq[:, 1:2], pq[:, 2:3]
        kx, ky, kz = pkT[0:1, :], pkT[1:2, :], pkT[2:3, :]
        dx = qx - kx
        dy = qy - ky
        dz = qz - kz
        d = (dx * dx + dy * dy) + dz * dz
        iota = lax.broadcasted_iota(jnp.int32, (Qb, Nk), 1)
        cols_i, cols_d = [], []
        for j in range(K):
            m = jnp.min(d, axis=1, keepdims=True)
            am = jnp.min(jnp.where(d == m, iota, Nk), axis=1, keepdims=True)
            cols_i.append(am)
            cols_d.append(m)
            if j < K - 1:
                d = jnp.where(iota == am, jnp.float32(jnp.inf), d)
        idx_ref[0] = jnp.concatenate(cols_i, axis=1) + b * Nk
        dst_ref[0] = jnp.concatenate(cols_d, axis=1)

    grid = (B, Nq // Qb)
    return pl.pallas_call(
        body,
        grid=grid,
        in_specs=[
            pl.BlockSpec((1, Qb, 3), lambda b, i: (b, i, 0)),
            pl.BlockSpec((1, 3, Nk), lambda b, i: (b, 0, 0)),
        ],
        out_specs=[
            pl.BlockSpec((1, Qb, K), lambda b, i: (b, i, 0)),
            pl.BlockSpec((1, Qb, K), lambda b, i: (b, i, 0)),
        ],
        out_shape=[
            jax.ShapeDtypeStruct((B, Nq, K), jnp.int32),
            jax.ShapeDtypeStruct((B, Nq, K), jnp.float32),
        ],
    )


def _knn(pq, pkT, K):
    B, Nq, _ = pq.shape
    Nk = pkT.shape[2]
    Qb = min(Nq, 512)
    return _knn_fn(B, Nq, Nk, K, Qb)(pq, pkT)


@functools.lru_cache(maxsize=None)
def _fps_fn(B, N):
    Ms = [N // 4, N // 16, N // 64, N // 256]

    def level(px, py, pz, M, npT_ref, np_ref):
        Ncur = px.shape[1]
        iota = lax.broadcasted_iota(jnp.int32, (B, Ncur), 1)
        iota3 = lax.broadcasted_iota(jnp.int32, (B, 3, M), 2)
        p0x, p0y, p0z = px[:, 0:1], py[:, 0:1], pz[:, 0:1]
        dx, dy, dz = px - p0x, py - p0y, pz - p0z
        mind0 = (dx * dx + dy * dy) + dz * dz
        cvec0 = jnp.concatenate(
            [p0x[:, None, :], p0y[:, None, :], p0z[:, None, :]], axis=1
        )
        tacc0 = cvec0 * (iota3 == 0).astype(jnp.float32)

        def step(i, carry):
            mind, tacc = carry
            m = jnp.max(mind, axis=1, keepdims=True)
            nxt = jnp.min(jnp.where(mind == m, iota, Ncur), axis=1, keepdims=True)
            sel = iota == nxt
            ptx = jnp.sum(jnp.where(sel, px, 0.0), axis=1, keepdims=True)
            pty = jnp.sum(jnp.where(sel, py, 0.0), axis=1, keepdims=True)
            ptz = jnp.sum(jnp.where(sel, pz, 0.0), axis=1, keepdims=True)
            ddx, ddy, ddz = px - ptx, py - pty, pz - ptz
            d = (ddx * ddx + ddy * ddy) + ddz * ddz
            cvec = jnp.concatenate(
                [ptx[:, None, :], pty[:, None, :], ptz[:, None, :]], axis=1
            )
            tacc = tacc + cvec * (iota3 == i).astype(jnp.float32)
            return jnp.minimum(mind, d), tacc

        _, tacc = lax.fori_loop(1, M, step, (mind0, tacc0))
        npT_ref[...] = tacc
        np_ref[...] = jnp.swapaxes(tacc, 1, 2)
        return tacc[:, 0, :], tacc[:, 1, :], tacc[:, 2, :]

    def body(pT_ref, o1T, o1, o2T, o2, o3T, o3, o4T, o4):
        pT = pT_ref[...]
        px, py, pz = pT[:, 0, :], pT[:, 1, :], pT[:, 2, :]
        px, py, pz = level(px, py, pz, Ms[0], o1T, o1)
        px, py, pz = level(px, py, pz, Ms[1], o2T, o2)
        px, py, pz = level(px, py, pz, Ms[2], o3T, o3)
        level(px, py, pz, Ms[3], o4T, o4)

    outs = []
    for M in Ms:
        outs.append(jax.ShapeDtypeStruct((B, 3, M), jnp.float32))
        outs.append(jax.ShapeDtypeStruct((B, M, 3), jnp.float32))
    return pl.pallas_call(body, out_shape=outs)


def _wspec(shape):
    n = len(shape)
    return pl.BlockSpec(shape, lambda i, _n=n: (0,) * _n)


@functools.lru_cache(maxsize=None)
def _ptb_pre_fn(BN, d, dpre):
    R = min(BN, 512)
    din = dpre if dpre is not None else d
    nw = 10 if dpre is not None else 8

    def body(x_ref, p_ref, *refs):
        ws = [r[...] for r in refs[:nw]]
        outs = refs[nw:]
        x = x_ref[...]
        i = 0
        if dpre is not None:
            x = x @ ws[0] + ws[1]
            i = 2
        Win, bin_, Wq, bq, Wk, bk, Wv, bv = ws[i : i + 8]
        h = x @ Win + bin_
        q = h @ Wq + bq
        kf = h @ Wk + bk
        vf = h @ Wv + bv
        tab = jnp.concatenate(
            [kf, vf, p_ref[...], jnp.zeros((R, 13), jnp.float32)], axis=1
        )
        outs[0][...] = q
        outs[1][...] = tab
        if dpre is not None:
            outs[2][...] = x

    def row(c):
        return pl.BlockSpec((R, c), lambda i: (i, 0))

    in_specs = [row(din), row(3)]
    if dpre is not None:
        in_specs += [_wspec((din, d)), _wspec((1, d))]
    in_specs += [_wspec((d, d)), _wspec((1, d))] * 4
    out_specs = [row(d), row(2 * d + 16)]
    out_shape = [
        jax.ShapeDtypeStruct((BN, d), jnp.float32),
        jax.ShapeDtypeStruct((BN, 2 * d + 16), jnp.float32),
    ]
    if dpre is not None:
        out_specs.append(row(d))
        out_shape.append(jax.ShapeDtypeStruct((BN, d), jnp.float32))
    return pl.pallas_call(
        body,
        grid=(BN // R,),
        in_specs=in_specs,
        out_specs=out_specs,
        out_shape=out_shape,
    )


@functools.lru_cache(maxsize=None)
def _ptb_post_fn(BN, d, K, dfin):
    Q = min(BN, 256)
    nw = 12 if dfin is not None else 10
    dout = dfin if dfin is not None else d

    def body(x_ref, p_ref, q_ref, kn_ref, vn_ref, pg_ref, *refs):
        ws = [r[...] for r in refs[:nw]]
        y_ref = refs[nw]
        P1, b1, P2, b2, A1, a1, A2, a2, Wout, bout = ws[:10]
        kn = kn_ref[...]
        vn = vn_ref[...]
        pg = pg_ref[...]
        p3 = p_ref[...][:, None, :]
        rel = (p3 - pg).reshape(Q * K, 3)
        pos = jnp.maximum(rel @ P1 + b1, 0.0) @ P2 + b2
        qv = q_ref[...]
        t = (qv[:, None, :] - kn).reshape(Q * K, d) + pos
        a = (jnp.maximum(t @ A1 + a1, 0.0) @ A2 + a2).reshape(Q, K, d)
        m = jnp.max(a, axis=1, keepdims=True)
        e = jnp.exp(a - m)
        s = jnp.sum(e, axis=1, keepdims=True)
        a = e / s
        pos3 = pos.reshape(Q, K, d)
        o = jnp.sum(a * (vn + pos3), axis=1)
        y = x_ref[...] + o @ Wout + bout
        if dfin is not None:
            y = y @ ws[10] + ws[11]
        y_ref[...] = y

    def row(c):
        return pl.BlockSpec((Q, c), lambda i: (i, 0))

    def row3(c):
        return pl.BlockSpec((Q, K, c), lambda i: (i, 0, 0))

    in_specs = [row(d), row(3), row(d), row3(d), row3(d), row3(3)]
    in_specs += [
        _wspec((3, d)), _wspec((1, d)),
        _wspec((d, d)), _wspec((1, d)),
        _wspec((d, d)), _wspec((1, d)),
        _wspec((d, d)), _wspec((1, d)),
        _wspec((d, d)), _wspec((1, d)),
    ]
    if dfin is not None:
        in_specs += [_wspec((d, dfin)), _wspec((1, dfin))]
    return pl.pallas_call(
        body,
        grid=(BN // Q,),
        in_specs=in_specs,
        out_specs=[row(dout)],
        out_shape=[jax.ShapeDtypeStruct((BN, dout), jnp.float32)],
    )


@functools.lru_cache(maxsize=None)
def _td_fn(BM, d, dout, K):
    Q = min(BM, 256)

    def body(np_ref, nx_ref, pg_ref, w1_ref, w2_ref, b_ref, o_ref):
        nx = nx_ref[...].reshape(Q * K, d)
        pg = pg_ref[...]
        rel = (np_ref[...][:, None, :] - pg).reshape(Q * K, 3)
        feat = nx @ w1_ref[...] + rel @ w2_ref[...] + b_ref[...]
        feat = jnp.maximum(feat, 0.0).reshape(Q, K, dout)
        o_ref[...] = jnp.max(feat, axis=1)

    return pl.pallas_call(
        body,
        grid=(BM // Q,),
        in_specs=[
            pl.BlockSpec((Q, 3), lambda i: (i, 0)),
            pl.BlockSpec((Q, K, d), lambda i: (i, 0, 0)),
            pl.BlockSpec((Q, K, 3), lambda i: (i, 0, 0)),
            _wspec((d, dout)),
            _wspec((3, dout)),
            _wspec((1, dout)),
        ],
        out_specs=[pl.BlockSpec((Q, dout), lambda i: (i, 0))],
        out_shape=[jax.ShapeDtypeStruct((BM, dout), jnp.float32)],
    )


@functools.lru_cache(maxsize=None)
def _tu_fn(BMf, do):
    Q = min(BMf, 256)

    def body(xs_ref, g_ref, dst_ref, w2_ref, b2_ref, o_ref):
        x2 = xs_ref[...] @ w2_ref[...] + b2_ref[...]
        w = 1.0 / (dst_ref[...] + 1e-8)
        w = w / jnp.sum(w, axis=1, keepdims=True)
        o = jnp.sum(g_ref[...] * w[:, :, None], axis=1)
        o_ref[...] = x2 + o

    return pl.pallas_call(
        body,
        grid=(BMf // Q,),
        in_specs=[
            pl.BlockSpec((Q, do), lambda i: (i, 0)),
            pl.BlockSpec((Q, 3, do), lambda i: (i, 0, 0)),
            pl.BlockSpec((Q, 3), lambda i: (i, 0)),
            _wspec((do, do)),
            _wspec((1, do)),
        ],
        out_specs=[pl.BlockSpec((Q, do), lambda i: (i, 0))],
        out_shape=[jax.ShapeDtypeStruct((BMf, do), jnp.float32)],
    )


@functools.lru_cache(maxsize=None)
def _linear_fn(BN, din, dout):
    R = min(BN, 512)

    def body(x_ref, w_ref, b_ref, o_ref):
        o_ref[...] = x_ref[...] @ w_ref[...] + b_ref[...]

    return pl.pallas_call(
        body,
        grid=(BN // R,),
        in_specs=[
            pl.BlockSpec((R, din), lambda i: (i, 0)),
            _wspec((din, dout)),
            _wspec((1, dout)),
        ],
        out_specs=[pl.BlockSpec((R, dout), lambda i: (i, 0))],
        out_shape=[jax.ShapeDtypeStruct((BN, dout), jnp.float32)],
    )


def _b(pr):
    return pr["b"].reshape(1, -1)


def _run_ptb(xf, pf, idxf, prm, d, BN, K, pre=None, fin=None):
    args = [xf, pf]
    if pre is not None:
        args += [pre["W"], _b(pre)]
    for n in ("lin_in", "q", "k", "v"):
        args += [prm[n]["W"], _b(prm[n])]
    dpre = pre["W"].shape[0] if pre is not None else None
    outs = _ptb_pre_fn(BN, d, dpre)(*args)
    q, tab = outs[0], outs[1]
    x_res = outs[2] if pre is not None else xf
    Dt = 2 * d + 16
    g = _gather_rows(tab, idxf).reshape(BN, K, Dt)
    kn = g[:, :, :d]
    vn = g[:, :, d : 2 * d]
    pg = g[:, :, 2 * d : 2 * d + 3]
    pargs = [x_res, pf, q, kn, vn, pg]
    for n in ("pos1", "pos2", "att1", "att2", "lin_out"):
        pargs += [prm[n]["W"], _b(prm[n])]
    dfin = None
    if fin is not None:
        pargs += [fin["W"], _b(fin)]
        dfin = fin["W"].shape[1]
    (y,) = _ptb_post_fn(BN, d, K, dfin)(*pargs)
    return y, x_res


def _run_td(xf_fine, pf_fine, np_l, idxf, prm, d, dout, BNf, BM, K=16):
    tab = jnp.concatenate(
        [xf_fine, pf_fine, jnp.zeros((BNf, 13), jnp.float32)], axis=1
    )
    g = _gather_rows(tab, idxf).reshape(BM, K, d + 16)
    nx = g[:, :, :d]
    pg = g[:, :, d : d + 3]
    W = prm["mlp"]["W"]
    (o,) = _td_fn(BM, d, dout, K)(
        np_l.reshape(BM, 3), nx, pg, W[:d], W[d:], _b(prm["mlp"])
    )
    return o


def _run_tu(xc_f, xs_f, dist, idxf, prm, dc, do, BMc, BMf):
    (x1,) = _linear_fn(BMc, dc, do)(xc_f, prm["lin1"]["W"], _b(prm["lin1"]))
    g = _gather_rows(x1, idxf).reshape(BMf, 3, do)
    (y,) = _tu_fn(BMf, do)(
        xs_f, g, dist.reshape(BMf, 3), prm["lin2"]["W"], _b(prm["lin2"])
    )
    return y


def kernel(x, p, params):
    B, N, CIN = x.shape
    Ns = [N, N // 4, N // 16, N // 64, N // 256]
    ds = [32, 64, 128, 256, 512]
    Ks = [min(16, n) for n in Ns]

    xf0 = x.reshape(B * N, CIN)
    pf = [p.reshape(B * N, 3)]
    pT0 = jnp.transpose(p, (0, 2, 1))

    fo = _fps_fn(B, N)(pT0)
    npT = [pT0, fo[0], fo[2], fo[4], fo[6]]
    np3 = [p, fo[1], fo[3], fo[5], fo[7]]
    for l in range(1, 5):
        pf.append(np3[l].reshape(B * Ns[l], 3))

    iself = []
    for l in range(5):
        idx, _ = _knn(np3[l], npT[l], Ks[l])
        iself.append(idx.reshape(-1))

    y, x_skip0 = _run_ptb(
        xf0, pf[0], iself[0], params["PTB0"], ds[0], B * Ns[0], Ks[0],
        pre=params["MLP0"],
    )
    skips = {0: y}
    xcur = y

    enc_names = [("TD1", "PTB1e"), ("TD2", "PTB2e"), ("TD3", "PTB3e"), ("TD4", "PTB4e")]
    for l, (tdn, ptbn) in enumerate(enc_names, start=1):
        itd, _ = _knn(np3[l], npT[l - 1], 16)
        xcur = _run_td(
            xcur, pf[l - 1], np3[l], itd.reshape(-1), params[tdn],
            ds[l - 1], ds[l], B * Ns[l - 1], B * Ns[l],
        )
        xcur, _ = _run_ptb(
            xcur, pf[l], iself[l], params[ptbn], ds[l], B * Ns[l], Ks[l]
        )
        if l < 4:
            skips[l] = xcur

    xcur, _ = _run_ptb(
        xcur, pf[4], iself[4], params["PTBm"], ds[4], B * Ns[4], Ks[4],
        pre=params["MLP1"],
    )

    dec_names = [("TU1", "PTB1d"), ("TU2", "PTB2d"), ("TU3", "PTB3d"), ("TU4", "PTB4d")]
    for i, (tun, ptbn) in enumerate(dec_names):
        lc = 4 - i
        lf = lc - 1
        itu, dtu = _knn(np3[lf], npT[lc], 3)
        xcur = _run_tu(
            xcur, skips[lf], dtu, itu.reshape(-1), params[tun],
            ds[lc], ds[lf], B * Ns[lc], B * Ns[lf],
        )
        fin = params["out"] if ptbn == "PTB4d" else None
        xcur, _ = _run_ptb(
            xcur, pf[lf], iself[lf], params[ptbn], ds[lf], B * Ns[lf], Ks[lf],
            fin=fin,
        )

    ncls = params["out"]["W"].shape[1]
    return xcur.reshape(B, N, ncls), p


# speedup vs baseline: 12.7170x; 1.2722x over previous
import functools

import jax
import jax.numpy as jnp
from jax import lax
from jax.experimental import pallas as pl
from jax.experimental.pallas import tpu as pltpu
from jax.experimental.pallas import tpu_sc as plsc


_SC_INFO = None


def _sc_info():
    global _SC_INFO
    if _SC_INFO is None:
        _SC_INFO = plsc.get_sparse_core_info()
    return _SC_INFO


@functools.lru_cache(maxsize=None)
def _gather_fn(R, D, Bn):
    info = _sc_info()
    NC, NS = info.num_cores, info.num_subcores
    NW = NC * NS
    bpw = Bn // NW
    c = 8
    for cand in range(min(128, bpw), 7, -8):
        if bpw % cand == 0 and cand * (D + 1) * 4 <= 480_000:
            c = cand
            break
    nch = bpw // c
    mesh = plsc.VectorSubcoreMesh(core_axis_name="c", subcore_axis_name="s")

    @functools.partial(
        pl.kernel,
        mesh=mesh,
        compiler_params=pltpu.CompilerParams(use_tc_tiling_on_sc=False),
        out_type=jax.ShapeDtypeStruct((Bn, D), jnp.float32),
        scratch_types=[
            pltpu.VMEM((c,), jnp.int32),
            pltpu.VMEM((c, D), jnp.float32),
            pltpu.SemaphoreType.DMA,
        ],
    )
    def k(table_hbm, idx_hbm, out_hbm, idx_v, rows_v, sem):
        wid = lax.axis_index("s") * NC + lax.axis_index("c")
        base = wid * bpw

        def step(j, carry):
            off = base + j * c
            pltpu.sync_copy(idx_hbm.at[pl.ds(off, c)], idx_v)
            pltpu.async_copy(table_hbm.at[idx_v], rows_v, sem).wait()
            pltpu.sync_copy(rows_v, out_hbm.at[pl.ds(off, c)])
            return carry

        lax.fori_loop(0, nch, step, 0)

    return k


def _gather_rows(table, idx):
    Bn0 = idx.shape[0]
    Bn = -(-Bn0 // 256) * 256
    if Bn != Bn0:
        idx = jnp.concatenate([idx, jnp.zeros((Bn - Bn0,), jnp.int32)])
    out = _gather_fn(table.shape[0], table.shape[1], Bn)(table, idx)
    return out[:Bn0]


@functools.lru_cache(maxsize=None)
def _knn_fn(B, Nq, Nk, K, Qb):
    def body(pq_ref, pkT_ref, idx_ref, dst_ref):
        b = pl.program_id(0)
        pq = pq_ref[0]
        pkT = pkT_ref[0]
        qx, qy, qz = pq[:, 0:1], pq[:, 1:2], pq[:, 2:3]
        kx, ky, kz = pkT[0:1, :], pkT[1:2, :], pkT[2:3, :]
        dx = qx - kx
        dy = qy - ky
        dz = qz - kz
        d = (dx * dx + dy * dy) + dz * dz
        iota = lax.broadcasted_iota(jnp.int32, (Qb, Nk), 1)
        cols_i, cols_d = [], []
        for j in range(K):
            m = jnp.min(d, axis=1, keepdims=True)
            am = jnp.min(jnp.where(d == m, iota, Nk), axis=1, keepdims=True)
            cols_i.append(am)
            cols_d.append(m)
            if j < K - 1:
                d = jnp.where(iota == am, jnp.float32(jnp.inf), d)
        idx_ref[0] = jnp.concatenate(cols_i, axis=1) + b * Nk
        dst_ref[0] = jnp.concatenate(cols_d, axis=1)

    grid = (B, Nq // Qb)
    return pl.pallas_call(
        body,
        grid=grid,
        in_specs=[
            pl.BlockSpec((1, Qb, 3), lambda b, i: (b, i, 0)),
            pl.BlockSpec((1, 3, Nk), lambda b, i: (b, 0, 0)),
        ],
        out_specs=[
            pl.BlockSpec((1, Qb, K), lambda b, i: (b, i, 0)),
            pl.BlockSpec((1, Qb, K), lambda b, i: (b, i, 0)),
        ],
        out_shape=[
            jax.ShapeDtypeStruct((B, Nq, K), jnp.int32),
            jax.ShapeDtypeStruct((B, Nq, K), jnp.float32),
        ],
    )


def _knn(pq, pkT, K):
    B, Nq, _ = pq.shape
    Nk = pkT.shape[2]
    Qb = min(Nq, 512)
    return _knn_fn(B, Nq, Nk, K, Qb)(pq, pkT)


@functools.lru_cache(maxsize=None)
def _fps_fn(B, N):
    Ms = [N // 4, N // 16, N // 64, N // 256]

    def level(px, py, pz, M, npT_ref, np_ref):
        Ncur = px.shape[1]
        iota = lax.broadcasted_iota(jnp.int32, (B, Ncur), 1)
        iota3 = lax.broadcasted_iota(jnp.int32, (B, 3, M), 2)
        p0x, p0y, p0z = px[:, 0:1], py[:, 0:1], pz[:, 0:1]
        dx, dy, dz = px - p0x, py - p0y, pz - p0z
        mind0 = (dx * dx + dy * dy) + dz * dz
        cvec0 = jnp.concatenate(
            [p0x[:, None, :], p0y[:, None, :], p0z[:, None, :]], axis=1
        )
        tacc0 = cvec0 * (iota3 == 0).astype(jnp.float32)

        def step(i, carry):
            mind, tacc = carry
            m = jnp.max(mind, axis=1, keepdims=True)
            nxt = jnp.min(jnp.where(mind == m, iota, Ncur), axis=1, keepdims=True)
            sel = iota == nxt
            ptx = jnp.sum(jnp.where(sel, px, 0.0), axis=1, keepdims=True)
            pty = jnp.sum(jnp.where(sel, py, 0.0), axis=1, keepdims=True)
            ptz = jnp.sum(jnp.where(sel, pz, 0.0), axis=1, keepdims=True)
            ddx, ddy, ddz = px - ptx, py - pty, pz - ptz
            d = (ddx * ddx + ddy * ddy) + ddz * ddz
            cvec = jnp.concatenate(
                [ptx[:, None, :], pty[:, None, :], ptz[:, None, :]], axis=1
            )
            tacc = tacc + cvec * (iota3 == i).astype(jnp.float32)
            return jnp.minimum(mind, d), tacc

        _, tacc = lax.fori_loop(1, M, step, (mind0, tacc0))
        npT_ref[...] = tacc
        np_ref[...] = jnp.swapaxes(tacc, 1, 2)
        return tacc[:, 0, :], tacc[:, 1, :], tacc[:, 2, :]

    def body(pT_ref, o1T, o1, o2T, o2, o3T, o3, o4T, o4):
        pT = pT_ref[...]
        px, py, pz = pT[:, 0, :], pT[:, 1, :], pT[:, 2, :]
        px, py, pz = level(px, py, pz, Ms[0], o1T, o1)
        px, py, pz = level(px, py, pz, Ms[1], o2T, o2)
        px, py, pz = level(px, py, pz, Ms[2], o3T, o3)
        level(px, py, pz, Ms[3], o4T, o4)

    outs = []
    for M in Ms:
        outs.append(jax.ShapeDtypeStruct((B, 3, M), jnp.float32))
        outs.append(jax.ShapeDtypeStruct((B, M, 3), jnp.float32))
    return pl.pallas_call(body, out_shape=outs)


def _wspec(shape):
    n = len(shape)
    return pl.BlockSpec(shape, lambda i, _n=n: (0,) * _n)


@functools.lru_cache(maxsize=None)
def _ptb_pre_fn(BN, d, dpre):
    R = min(BN, 512)
    din = dpre if dpre is not None else d
    nw = 10 if dpre is not None else 8

    def body(x_ref, p_ref, *refs):
        ws = [r[...] for r in refs[:nw]]
        outs = refs[nw:]
        x = x_ref[...]
        i = 0
        if dpre is not None:
            x = x @ ws[0] + ws[1]
            i = 2
        Win, bin_, Wq, bq, Wk, bk, Wv, bv = ws[i : i + 8]
        h = x @ Win + bin_
        q = h @ Wq + bq
        kf = h @ Wk + bk
        vf = h @ Wv + bv
        tab = jnp.concatenate(
            [kf, vf, p_ref[...], jnp.zeros((R, 13), jnp.float32)], axis=1
        )
        outs[0][...] = q
        outs[1][...] = tab
        if dpre is not None:
            outs[2][...] = x

    def row(c):
        return pl.BlockSpec((R, c), lambda i: (i, 0))

    in_specs = [row(din), row(3)]
    if dpre is not None:
        in_specs += [_wspec((din, d)), _wspec((1, d))]
    in_specs += [_wspec((d, d)), _wspec((1, d))] * 4
    out_specs = [row(d), row(2 * d + 16)]
    out_shape = [
        jax.ShapeDtypeStruct((BN, d), jnp.float32),
        jax.ShapeDtypeStruct((BN, 2 * d + 16), jnp.float32),
    ]
    if dpre is not None:
        out_specs.append(row(d))
        out_shape.append(jax.ShapeDtypeStruct((BN, d), jnp.float32))
    return pl.pallas_call(
        body,
        grid=(BN // R,),
        in_specs=in_specs,
        out_specs=out_specs,
        out_shape=out_shape,
    )


@functools.lru_cache(maxsize=None)
def _ptb_post_fn(BN, d, K, dfin, BNtab, Q):
    nw = 12 if dfin is not None else 10
    dout = dfin if dfin is not None else d
    Dt = 2 * d + 16

    def body(x_ref, p_ref, q_ref, *refs):
        if BNtab is None:
            (g_ref,) = refs[:1]
            refs = refs[1:]
            g = g_ref[...]
        else:
            idx_ref, tab_ref = refs[:2]
            refs = refs[2:]
            iota = lax.broadcasted_iota(jnp.int32, (Q * K, BNtab), 1)
            onehot = (iota == idx_ref[...]).astype(jnp.float32)
            g = (onehot @ tab_ref[...]).reshape(Q, K, Dt)
        ws = [r[...] for r in refs[:nw]]
        y_ref = refs[nw]
        P1, b1, P2, b2, A1, a1, A2, a2, Wout, bout = ws[:10]
        kn = g[:, :, :d]
        vn = g[:, :, d : 2 * d]
        pg = g[:, :, 2 * d : 2 * d + 3]
        p3 = p_ref[...][:, None, :]
        rel = (p3 - pg).reshape(Q * K, 3)
        pos = jnp.maximum(rel @ P1 + b1, 0.0) @ P2 + b2
        qv = q_ref[...]
        t = (qv[:, None, :] - kn).reshape(Q * K, d) + pos
        a = (jnp.maximum(t @ A1 + a1, 0.0) @ A2 + a2).reshape(Q, K, d)
        m = jnp.max(a, axis=1, keepdims=True)
        e = jnp.exp(a - m)
        s = jnp.sum(e, axis=1, keepdims=True)
        a = e / s
        pos3 = pos.reshape(Q, K, d)
        o = jnp.sum(a * (vn + pos3), axis=1)
        y = x_ref[...] + o @ Wout + bout
        if dfin is not None:
            y = y @ ws[10] + ws[11]
        y_ref[...] = y

    def row(c, dt=None):
        return pl.BlockSpec((Q, c), lambda i: (i, 0))

    in_specs = [row(d), row(3), row(d)]
    if BNtab is None:
        in_specs += [pl.BlockSpec((Q, K, Dt), lambda i: (i, 0, 0))]
    else:
        in_specs += [
            pl.BlockSpec((Q * K, 1), lambda i: (i, 0)),
            _wspec((BNtab, Dt)),
        ]
    in_specs += [
        _wspec((3, d)), _wspec((1, d)),
        _wspec((d, d)), _wspec((1, d)),
        _wspec((d, d)), _wspec((1, d)),
        _wspec((d, d)), _wspec((1, d)),
        _wspec((d, d)), _wspec((1, d)),
    ]
    if dfin is not None:
        in_specs += [_wspec((d, dfin)), _wspec((1, dfin))]
    return pl.pallas_call(
        body,
        grid=(BN // Q,),
        in_specs=in_specs,
        out_specs=[row(dout)],
        out_shape=[jax.ShapeDtypeStruct((BN, dout), jnp.float32)],
    )


@functools.lru_cache(maxsize=None)
def _td_fn(BM, d, dout, K, BNtab, Q):
    Dt = d + 16

    def body(np_ref, *refs):
        if BNtab is None:
            (g_ref,) = refs[:1]
            refs = refs[1:]
            g = g_ref[...]
        else:
            idx_ref, tab_ref = refs[:2]
            refs = refs[2:]
            iota = lax.broadcasted_iota(jnp.int32, (Q * K, BNtab), 1)
            onehot = (iota == idx_ref[...]).astype(jnp.float32)
            g = (onehot @ tab_ref[...]).reshape(Q, K, Dt)
        w1_ref, w2_ref, b_ref, o_ref = refs
        nx = g[:, :, :d].reshape(Q * K, d)
        pg = g[:, :, d : d + 3]
        rel = (np_ref[...][:, None, :] - pg).reshape(Q * K, 3)
        feat = nx @ w1_ref[...] + rel @ w2_ref[...] + b_ref[...]
        feat = jnp.maximum(feat, 0.0).reshape(Q, K, dout)
        o_ref[...] = jnp.max(feat, axis=1)

    in_specs = [pl.BlockSpec((Q, 3), lambda i: (i, 0))]
    if BNtab is None:
        in_specs += [pl.BlockSpec((Q, K, Dt), lambda i: (i, 0, 0))]
    else:
        in_specs += [
            pl.BlockSpec((Q * K, 1), lambda i: (i, 0)),
            _wspec((BNtab, Dt)),
        ]
    in_specs += [_wspec((d, dout)), _wspec((3, dout)), _wspec((1, dout))]
    return pl.pallas_call(
        body,
        grid=(BM // Q,),
        in_specs=in_specs,
        out_specs=[pl.BlockSpec((Q, dout), lambda i: (i, 0))],
        out_shape=[jax.ShapeDtypeStruct((BM, dout), jnp.float32)],
    )


@functools.lru_cache(maxsize=None)
def _tu_fn(BMf, do, BNtab, Q):
    def body(xs_ref, dst_ref, *refs):
        w2_ref, b2_ref, o_ref = refs[-3:]
        x2 = xs_ref[...] @ w2_ref[...] + b2_ref[...]
        w = 1.0 / (dst_ref[...] + 1e-8)
        w = w / jnp.sum(w, axis=1, keepdims=True)
        if BNtab is None:
            g = refs[0][...]
            o = jnp.sum(g * w[:, :, None], axis=1)
        else:
            idx_ref, tab_ref = refs[:2]
            idx = idx_ref[...]
            tab = tab_ref[...]
            iota = lax.broadcasted_iota(jnp.int32, (Q, BNtab), 1)
            o = jnp.zeros((Q, do), jnp.float32)
            for j in range(3):
                oh = (iota == idx[:, j : j + 1]).astype(jnp.float32)
                o = o + (oh @ tab) * w[:, j : j + 1]
        o_ref[...] = x2 + o

    in_specs = [
        pl.BlockSpec((Q, do), lambda i: (i, 0)),
        pl.BlockSpec((Q, 3), lambda i: (i, 0)),
    ]
    if BNtab is None:
        in_specs += [pl.BlockSpec((Q, 3, do), lambda i: (i, 0, 0))]
    else:
        in_specs += [pl.BlockSpec((Q, 3), lambda i: (i, 0)), _wspec((BNtab, do))]
    in_specs += [_wspec((do, do)), _wspec((1, do))]
    return pl.pallas_call(
        body,
        grid=(BMf // Q,),
        in_specs=in_specs,
        out_specs=[pl.BlockSpec((Q, do), lambda i: (i, 0))],
        out_shape=[jax.ShapeDtypeStruct((BMf, do), jnp.float32)],
    )


@functools.lru_cache(maxsize=None)
def _linear_fn(BN, din, dout):
    R = min(BN, 512)

    def body(x_ref, w_ref, b_ref, o_ref):
        o_ref[...] = x_ref[...] @ w_ref[...] + b_ref[...]

    return pl.pallas_call(
        body,
        grid=(BN // R,),
        in_specs=[
            pl.BlockSpec((R, din), lambda i: (i, 0)),
            _wspec((din, dout)),
            _wspec((1, dout)),
        ],
        out_specs=[pl.BlockSpec((R, dout), lambda i: (i, 0))],
        out_shape=[jax.ShapeDtypeStruct((BN, dout), jnp.float32)],
    )


def _b(pr):
    return pr["b"].reshape(1, -1)


_SC_MIN_TABLE_ROWS = 2048


def _pick_q(BN, K, BNtab):
    q = min(BN, 256)
    if BNtab is not None:
        while q > 8 and q * K * BNtab * 4 > 4 * 1024 * 1024:
            q //= 2
    return q


def _run_ptb(xf, pf, idxf, prm, d, BN, K, pre=None, fin=None):
    args = [xf, pf]
    if pre is not None:
        args += [pre["W"], _b(pre)]
    for n in ("lin_in", "q", "k", "v"):
        args += [prm[n]["W"], _b(prm[n])]
    dpre = pre["W"].shape[0] if pre is not None else None
    outs = _ptb_pre_fn(BN, d, dpre)(*args)
    q, tab = outs[0], outs[1]
    x_res = outs[2] if pre is not None else xf
    Dt = 2 * d + 16
    use_sc = tab.shape[0] >= _SC_MIN_TABLE_ROWS
    if use_sc:
        g = _gather_rows(tab, idxf).reshape(BN, K, Dt)
        gargs, BNtab = [g], None
    else:
        gargs, BNtab = [idxf.reshape(BN * K, 1), tab], tab.shape[0]
    pargs = [x_res, pf, q] + gargs
    for n in ("pos1", "pos2", "att1", "att2", "lin_out"):
        pargs += [prm[n]["W"], _b(prm[n])]
    dfin = None
    if fin is not None:
        pargs += [fin["W"], _b(fin)]
        dfin = fin["W"].shape[1]
    (y,) = _ptb_post_fn(BN, d, K, dfin, BNtab, _pick_q(BN, K, BNtab))(*pargs)
    return y, x_res


def _run_td(xf_fine, pf_fine, np_l, idxf, prm, d, dout, BNf, BM, K=16):
    tab = jnp.concatenate(
        [xf_fine, pf_fine, jnp.zeros((BNf, 13), jnp.float32)], axis=1
    )
    use_sc = BNf >= _SC_MIN_TABLE_ROWS
    if use_sc:
        g = _gather_rows(tab, idxf).reshape(BM, K, d + 16)
        gargs, BNtab = [g], None
    else:
        gargs, BNtab = [idxf.reshape(BM * K, 1), tab], BNf
    W = prm["mlp"]["W"]
    (o,) = _td_fn(BM, d, dout, K, BNtab, _pick_q(BM, K, BNtab))(
        np_l.reshape(BM, 3), *gargs, W[:d], W[d:], _b(prm["mlp"])
    )
    return o


def _run_tu(xc_f, xs_f, dist, idxf, prm, dc, do, BMc, BMf):
    (x1,) = _linear_fn(BMc, dc, do)(xc_f, prm["lin1"]["W"], _b(prm["lin1"]))
    use_sc = BMc >= _SC_MIN_TABLE_ROWS
    if use_sc:
        g = _gather_rows(x1, idxf).reshape(BMf, 3, do)
        gargs, BNtab = [g], None
    else:
        gargs, BNtab = [idxf.reshape(BMf, 3), x1], BMc
    (y,) = _tu_fn(BMf, do, BNtab, _pick_q(BMf, 3, BNtab))(
        xs_f, dist.reshape(BMf, 3), *gargs, prm["lin2"]["W"], _b(prm["lin2"])
    )
    return y


def kernel(x, p, params):
    B, N, CIN = x.shape
    Ns = [N, N // 4, N // 16, N // 64, N // 256]
    ds = [32, 64, 128, 256, 512]
    Ks = [min(16, n) for n in Ns]

    xf0 = x.reshape(B * N, CIN)
    pf = [p.reshape(B * N, 3)]
    pT0 = jnp.transpose(p, (0, 2, 1))

    fo = _fps_fn(B, N)(pT0)
    npT = [pT0, fo[0], fo[2], fo[4], fo[6]]
    np3 = [p, fo[1], fo[3], fo[5], fo[7]]
    for l in range(1, 5):
        pf.append(np3[l].reshape(B * Ns[l], 3))

    iself = []
    for l in range(5):
        idx, _ = _knn(np3[l], npT[l], Ks[l])
        iself.append(idx.reshape(-1))

    y, x_skip0 = _run_ptb(
        xf0, pf[0], iself[0], params["PTB0"], ds[0], B * Ns[0], Ks[0],
        pre=params["MLP0"],
    )
    skips = {0: y}
    xcur = y

    enc_names = [("TD1", "PTB1e"), ("TD2", "PTB2e"), ("TD3", "PTB3e"), ("TD4", "PTB4e")]
    for l, (tdn, ptbn) in enumerate(enc_names, start=1):
        itd, _ = _knn(np3[l], npT[l - 1], 16)
        xcur = _run_td(
            xcur, pf[l - 1], np3[l], itd.reshape(-1), params[tdn],
            ds[l - 1], ds[l], B * Ns[l - 1], B * Ns[l],
        )
        xcur, _ = _run_ptb(
            xcur, pf[l], iself[l], params[ptbn], ds[l], B * Ns[l], Ks[l]
        )
        if l < 4:
            skips[l] = xcur

    xcur, _ = _run_ptb(
        xcur, pf[4], iself[4], params["PTBm"], ds[4], B * Ns[4], Ks[4],
        pre=params["MLP1"],
    )

    dec_names = [("TU1", "PTB1d"), ("TU2", "PTB2d"), ("TU3", "PTB3d"), ("TU4", "PTB4d")]
    for i, (tun, ptbn) in enumerate(dec_names):
        lc = 4 - i
        lf = lc - 1
        itu, dtu = _knn(np3[lf], npT[lc], 3)
        xcur = _run_tu(
            xcur, skips[lf], dtu, itu.reshape(-1), params[tun],
            ds[lc], ds[lf], B * Ns[lc], B * Ns[lf],
        )
        fin = params["out"] if ptbn == "PTB4d" else None
        xcur, _ = _run_ptb(
            xcur, pf[lf], iself[lf], params[ptbn], ds[lf], B * Ns[lf], Ks[lf],
            fin=fin,
        )

    ncls = params["out"]["W"].shape[1]
    return xcur.reshape(B, N, ncls), p


# speedup vs baseline: 13.2230x; 1.0398x over previous
import functools

import jax
import jax.numpy as jnp
from jax import lax
from jax.experimental import pallas as pl
from jax.experimental.pallas import tpu as pltpu
from jax.experimental.pallas import tpu_sc as plsc


_SC_INFO = None


def _sc_info():
    global _SC_INFO
    if _SC_INFO is None:
        _SC_INFO = plsc.get_sparse_core_info()
    return _SC_INFO


@functools.lru_cache(maxsize=None)
def _gather_fn(R, D, Bn):
    info = _sc_info()
    NC, NS = info.num_cores, info.num_subcores
    NW = NC * NS
    bpw = Bn // NW
    c = 8
    for cand in range(min(128, bpw), 7, -8):
        if bpw % cand == 0 and 8 * cand * D + 4 * bpw <= 440_000:
            c = cand
            break
    nch = bpw // c
    mesh = plsc.VectorSubcoreMesh(core_axis_name="c", subcore_axis_name="s")

    @functools.partial(
        pl.kernel,
        mesh=mesh,
        compiler_params=pltpu.CompilerParams(use_tc_tiling_on_sc=False),
        out_type=jax.ShapeDtypeStruct((Bn, D), jnp.float32),
        scratch_types=[
            pltpu.VMEM((bpw,), jnp.int32),
            pltpu.VMEM((c, D), jnp.float32),
            pltpu.VMEM((c, D), jnp.float32),
            pltpu.SemaphoreType.DMA,
            pltpu.SemaphoreType.DMA,
            pltpu.SemaphoreType.DMA,
            pltpu.SemaphoreType.DMA,
        ],
    )
    def k(table_hbm, idx_hbm, out_hbm, idx_all, r0, r1, g0, g1, w0, w1):
        wid = lax.axis_index("s") * NC + lax.axis_index("c")
        base = wid * bpw
        rows = (r0, r1)
        gsem = (g0, g1)
        wsem = (w0, w1)
        pltpu.sync_copy(idx_hbm.at[pl.ds(base, bpw)], idx_all)
        gcp = [None, None]
        wcp = [None, None]
        for j in range(nch):
            b = j & 1
            if j >= 2:
                wcp[b].wait()
            gcp[b] = pltpu.async_copy(
                table_hbm.at[idx_all.at[pl.ds(j * c, c)]], rows[b], gsem[b]
            )
            if j >= 1:
                pb = (j - 1) & 1
                gcp[pb].wait()
                wcp[pb] = pltpu.async_copy(
                    rows[pb], out_hbm.at[pl.ds(base + (j - 1) * c, c)], wsem[pb]
                )
        lb = (nch - 1) & 1
        gcp[lb].wait()
        wcp[lb] = pltpu.async_copy(
            rows[lb], out_hbm.at[pl.ds(base + (nch - 1) * c, c)], wsem[lb]
        )
        if nch >= 2:
            wcp[(nch - 2) & 1].wait()
        wcp[lb].wait()

    return k


def _gather_rows(table, idx):
    Bn0 = idx.shape[0]
    Bn = -(-Bn0 // 256) * 256
    if Bn != Bn0:
        idx = jnp.concatenate([idx, jnp.zeros((Bn - Bn0,), jnp.int32)])
    out = _gather_fn(table.shape[0], table.shape[1], Bn)(table, idx)
    return out[:Bn0]


@functools.lru_cache(maxsize=None)
def _knn_fn(B, Nq, Nk, K, Qb):
    def body(pq_ref, pkT_ref, idx_ref, dst_ref):
        b = pl.program_id(0)
        pq = pq_ref[0]
        pkT = pkT_ref[0]
        qx, qy, qz = pq[:, 0:1], pq[:, 1:2], pq[:, 2:3]
        kx, ky, kz = pkT[0:1, :], pkT[1:2, :], pkT[2:3, :]
        dx = qx - kx
        dy = qy - ky
        dz = qz - kz
        d = (dx * dx + dy * dy) + dz * dz
        iota = lax.broadcasted_iota(jnp.int32, (Qb, Nk), 1)
        cols_i, cols_d = [], []
        for j in range(K):
            m = jnp.min(d, axis=1, keepdims=True)
            am = jnp.min(jnp.where(d == m, iota, Nk), axis=1, keepdims=True)
            cols_i.append(am)
            cols_d.append(m)
            if j < K - 1:
                d = jnp.where(iota == am, jnp.float32(jnp.inf), d)
        idx_ref[0] = jnp.concatenate(cols_i, axis=1) + b * Nk
        dst_ref[0] = jnp.concatenate(cols_d, axis=1)

    grid = (B, Nq // Qb)
    return pl.pallas_call(
        body,
        grid=grid,
        in_specs=[
            pl.BlockSpec((1, Qb, 3), lambda b, i: (b, i, 0)),
            pl.BlockSpec((1, 3, Nk), lambda b, i: (b, 0, 0)),
        ],
        out_specs=[
            pl.BlockSpec((1, Qb, K), lambda b, i: (b, i, 0)),
            pl.BlockSpec((1, Qb, K), lambda b, i: (b, i, 0)),
        ],
        out_shape=[
            jax.ShapeDtypeStruct((B, Nq, K), jnp.int32),
            jax.ShapeDtypeStruct((B, Nq, K), jnp.float32),
        ],
    )


def _knn(pq, pkT, K):
    B, Nq, _ = pq.shape
    Nk = pkT.shape[2]
    Qb = min(Nq, 512)
    return _knn_fn(B, Nq, Nk, K, Qb)(pq, pkT)


@functools.lru_cache(maxsize=None)
def _fps_fn(B, N):
    Ms = [N // 4, N // 16, N // 64, N // 256]

    def level(px, py, pz, M, npT_ref, np_ref):
        Ncur = px.shape[1]
        iota = lax.broadcasted_iota(jnp.int32, (B, Ncur), 1)
        iota3 = lax.broadcasted_iota(jnp.int32, (B, 3, M), 2)
        p0x, p0y, p0z = px[:, 0:1], py[:, 0:1], pz[:, 0:1]
        dx, dy, dz = px - p0x, py - p0y, pz - p0z
        mind0 = (dx * dx + dy * dy) + dz * dz
        cvec0 = jnp.concatenate(
            [p0x[:, None, :], p0y[:, None, :], p0z[:, None, :]], axis=1
        )
        tacc0 = cvec0 * (iota3 == 0).astype(jnp.float32)

        def step(i, carry):
            mind, tacc = carry
            m = jnp.max(mind, axis=1, keepdims=True)
            nxt = jnp.min(jnp.where(mind == m, iota, Ncur), axis=1, keepdims=True)
            sel = iota == nxt
            ptx = jnp.sum(jnp.where(sel, px, 0.0), axis=1, keepdims=True)
            pty = jnp.sum(jnp.where(sel, py, 0.0), axis=1, keepdims=True)
            ptz = jnp.sum(jnp.where(sel, pz, 0.0), axis=1, keepdims=True)
            ddx, ddy, ddz = px - ptx, py - pty, pz - ptz
            d = (ddx * ddx + ddy * ddy) + ddz * ddz
            cvec = jnp.concatenate(
                [ptx[:, None, :], pty[:, None, :], ptz[:, None, :]], axis=1
            )
            tacc = tacc + cvec * (iota3 == i).astype(jnp.float32)
            return jnp.minimum(mind, d), tacc

        _, tacc = lax.fori_loop(1, M, step, (mind0, tacc0))
        npT_ref[...] = tacc
        np_ref[...] = jnp.swapaxes(tacc, 1, 2)
        return tacc[:, 0, :], tacc[:, 1, :], tacc[:, 2, :]

    def body(pT_ref, o1T, o1, o2T, o2, o3T, o3, o4T, o4):
        pT = pT_ref[...]
        px, py, pz = pT[:, 0, :], pT[:, 1, :], pT[:, 2, :]
        px, py, pz = level(px, py, pz, Ms[0], o1T, o1)
        px, py, pz = level(px, py, pz, Ms[1], o2T, o2)
        px, py, pz = level(px, py, pz, Ms[2], o3T, o3)
        level(px, py, pz, Ms[3], o4T, o4)

    outs = []
    for M in Ms:
        outs.append(jax.ShapeDtypeStruct((B, 3, M), jnp.float32))
        outs.append(jax.ShapeDtypeStruct((B, M, 3), jnp.float32))
    return pl.pallas_call(body, out_shape=outs)


def _wspec(shape):
    n = len(shape)
    return pl.BlockSpec(shape, lambda i, _n=n: (0,) * _n)


@functools.lru_cache(maxsize=None)
def _ptb_pre_fn(BN, d, dpre):
    R = min(BN, 512)
    din = dpre if dpre is not None else d
    nw = 10 if dpre is not None else 8

    def body(x_ref, p_ref, *refs):
        ws = [r[...] for r in refs[:nw]]
        outs = refs[nw:]
        x = x_ref[...]
        i = 0
        if dpre is not None:
            x = x @ ws[0] + ws[1]
            i = 2
        Win, bin_, Wq, bq, Wk, bk, Wv, bv = ws[i : i + 8]
        h = x @ Win + bin_
        q = h @ Wq + bq
        kf = h @ Wk + bk
        vf = h @ Wv + bv
        tab = jnp.concatenate(
            [kf, vf, p_ref[...], jnp.zeros((R, 13), jnp.float32)], axis=1
        )
        outs[0][...] = q
        outs[1][...] = tab
        if dpre is not None:
            outs[2][...] = x

    def row(c):
        return pl.BlockSpec((R, c), lambda i: (i, 0))

    in_specs = [row(din), row(3)]
    if dpre is not None:
        in_specs += [_wspec((din, d)), _wspec((1, d))]
    in_specs += [_wspec((d, d)), _wspec((1, d))] * 4
    out_specs = [row(d), row(2 * d + 16)]
    out_shape = [
        jax.ShapeDtypeStruct((BN, d), jnp.float32),
        jax.ShapeDtypeStruct((BN, 2 * d + 16), jnp.float32),
    ]
    if dpre is not None:
        out_specs.append(row(d))
        out_shape.append(jax.ShapeDtypeStruct((BN, d), jnp.float32))
    return pl.pallas_call(
        body,
        grid=(BN // R,),
        in_specs=in_specs,
        out_specs=out_specs,
        out_shape=out_shape,
    )


@functools.lru_cache(maxsize=None)
def _ptb_post_fn(BN, d, K, dfin, BNtab, Q):
    nw = 12 if dfin is not None else 10
    dout = dfin if dfin is not None else d
    Dt = 2 * d + 16

    def body(x_ref, p_ref, q_ref, *refs):
        if BNtab is None:
            (g_ref,) = refs[:1]
            refs = refs[1:]
            g = g_ref[...]
        else:
            idx_ref, tab_ref = refs[:2]
            refs = refs[2:]
            iota = lax.broadcasted_iota(jnp.int32, (Q * K, BNtab), 1)
            onehot = (iota == idx_ref[...]).astype(jnp.float32)
            g = (onehot @ tab_ref[...]).reshape(Q, K, Dt)
        ws = [r[...] for r in refs[:nw]]
        y_ref = refs[nw]
        P1, b1, P2, b2, A1, a1, A2, a2, Wout, bout = ws[:10]
        kn = g[:, :, :d]
        vn = g[:, :, d : 2 * d]
        pg = g[:, :, 2 * d : 2 * d + 3]
        p3 = p_ref[...][:, None, :]
        rel = (p3 - pg).reshape(Q * K, 3)
        pos = jnp.maximum(rel @ P1 + b1, 0.0) @ P2 + b2
        qv = q_ref[...]
        t = (qv[:, None, :] - kn).reshape(Q * K, d) + pos
        a = (jnp.maximum(t @ A1 + a1, 0.0) @ A2 + a2).reshape(Q, K, d)
        m = jnp.max(a, axis=1, keepdims=True)
        e = jnp.exp(a - m)
        s = jnp.sum(e, axis=1, keepdims=True)
        a = e / s
        pos3 = pos.reshape(Q, K, d)
        o = jnp.sum(a * (vn + pos3), axis=1)
        y = x_ref[...] + o @ Wout + bout
        if dfin is not None:
            y = y @ ws[10] + ws[11]
        y_ref[...] = y

    def row(c, dt=None):
        return pl.BlockSpec((Q, c), lambda i: (i, 0))

    in_specs = [row(d), row(3), row(d)]
    if BNtab is None:
        in_specs += [pl.BlockSpec((Q, K, Dt), lambda i: (i, 0, 0))]
    else:
        in_specs += [
            pl.BlockSpec((Q * K, 1), lambda i: (i, 0)),
            _wspec((BNtab, Dt)),
        ]
    in_specs += [
        _wspec((3, d)), _wspec((1, d)),
        _wspec((d, d)), _wspec((1, d)),
        _wspec((d, d)), _wspec((1, d)),
        _wspec((d, d)), _wspec((1, d)),
        _wspec((d, d)), _wspec((1, d)),
    ]
    if dfin is not None:
        in_specs += [_wspec((d, dfin)), _wspec((1, dfin))]
    return pl.pallas_call(
        body,
        grid=(BN // Q,),
        in_specs=in_specs,
        out_specs=[row(dout)],
        out_shape=[jax.ShapeDtypeStruct((BN, dout), jnp.float32)],
    )


@functools.lru_cache(maxsize=None)
def _td_fn(BM, d, dout, K, BNtab, Q):
    Dt = d + 16

    def body(np_ref, *refs):
        if BNtab is None:
            (g_ref,) = refs[:1]
            refs = refs[1:]
            g = g_ref[...]
        else:
            idx_ref, tab_ref = refs[:2]
            refs = refs[2:]
            iota = lax.broadcasted_iota(jnp.int32, (Q * K, BNtab), 1)
            onehot = (iota == idx_ref[...]).astype(jnp.float32)
            g = (onehot @ tab_ref[...]).reshape(Q, K, Dt)
        w1_ref, w2_ref, b_ref, o_ref = refs
        nx = g[:, :, :d].reshape(Q * K, d)
        pg = g[:, :, d : d + 3]
        rel = (np_ref[...][:, None, :] - pg).reshape(Q * K, 3)
        feat = nx @ w1_ref[...] + rel @ w2_ref[...] + b_ref[...]
        feat = jnp.maximum(feat, 0.0).reshape(Q, K, dout)
        o_ref[...] = jnp.max(feat, axis=1)

    in_specs = [pl.BlockSpec((Q, 3), lambda i: (i, 0))]
    if BNtab is None:
        in_specs += [pl.BlockSpec((Q, K, Dt), lambda i: (i, 0, 0))]
    else:
        in_specs += [
            pl.BlockSpec((Q * K, 1), lambda i: (i, 0)),
            _wspec((BNtab, Dt)),
        ]
    in_specs += [_wspec((d, dout)), _wspec((3, dout)), _wspec((1, dout))]
    return pl.pallas_call(
        body,
        grid=(BM // Q,),
        in_specs=in_specs,
        out_specs=[pl.BlockSpec((Q, dout), lambda i: (i, 0))],
        out_shape=[jax.ShapeDtypeStruct((BM, dout), jnp.float32)],
    )


@functools.lru_cache(maxsize=None)
def _tu_fn(BMf, do, BNtab, Q):
    def body(xs_ref, dst_ref, *refs):
        w2_ref, b2_ref, o_ref = refs[-3:]
        x2 = xs_ref[...] @ w2_ref[...] + b2_ref[...]
        w = 1.0 / (dst_ref[...] + 1e-8)
        w = w / jnp.sum(w, axis=1, keepdims=True)
        if BNtab is None:
            g = refs[0][...]
            o = jnp.sum(g * w[:, :, None], axis=1)
        else:
            idx_ref, tab_ref = refs[:2]
            idx = idx_ref[...]
            tab = tab_ref[...]
            iota = lax.broadcasted_iota(jnp.int32, (Q, BNtab), 1)
            o = jnp.zeros((Q, do), jnp.float32)
            for j in range(3):
                oh = (iota == idx[:, j : j + 1]).astype(jnp.float32)
                o = o + (oh @ tab) * w[:, j : j + 1]
        o_ref[...] = x2 + o

    in_specs = [
        pl.BlockSpec((Q, do), lambda i: (i, 0)),
        pl.BlockSpec((Q, 3), lambda i: (i, 0)),
    ]
    if BNtab is None:
        in_specs += [pl.BlockSpec((Q, 3, do), lambda i: (i, 0, 0))]
    else:
        in_specs += [pl.BlockSpec((Q, 3), lambda i: (i, 0)), _wspec((BNtab, do))]
    in_specs += [_wspec((do, do)), _wspec((1, do))]
    return pl.pallas_call(
        body,
        grid=(BMf // Q,),
        in_specs=in_specs,
        out_specs=[pl.BlockSpec((Q, do), lambda i: (i, 0))],
        out_shape=[jax.ShapeDtypeStruct((BMf, do), jnp.float32)],
    )


@functools.lru_cache(maxsize=None)
def _linear_fn(BN, din, dout):
    R = min(BN, 512)

    def body(x_ref, w_ref, b_ref, o_ref):
        o_ref[...] = x_ref[...] @ w_ref[...] + b_ref[...]

    return pl.pallas_call(
        body,
        grid=(BN // R,),
        in_specs=[
            pl.BlockSpec((R, din), lambda i: (i, 0)),
            _wspec((din, dout)),
            _wspec((1, dout)),
        ],
        out_specs=[pl.BlockSpec((R, dout), lambda i: (i, 0))],
        out_shape=[jax.ShapeDtypeStruct((BN, dout), jnp.float32)],
    )


def _b(pr):
    return pr["b"].reshape(1, -1)


_SC_MIN_TABLE_ROWS = 2048


def _pick_q(BN, K, BNtab):
    q = min(BN, 512)
    if BNtab is not None:
        while q > 8 and q * K * BNtab * 4 > 4 * 1024 * 1024:
            q //= 2
    return q


def _run_ptb(xf, pf, idxf, prm, d, BN, K, pre=None, fin=None):
    args = [xf, pf]
    if pre is not None:
        args += [pre["W"], _b(pre)]
    for n in ("lin_in", "q", "k", "v"):
        args += [prm[n]["W"], _b(prm[n])]
    dpre = pre["W"].shape[0] if pre is not None else None
    outs = _ptb_pre_fn(BN, d, dpre)(*args)
    q, tab = outs[0], outs[1]
    x_res = outs[2] if pre is not None else xf
    Dt = 2 * d + 16
    use_sc = tab.shape[0] >= _SC_MIN_TABLE_ROWS
    if use_sc:
        g = _gather_rows(tab, idxf).reshape(BN, K, Dt)
        gargs, BNtab = [g], None
    else:
        gargs, BNtab = [idxf.reshape(BN * K, 1), tab], tab.shape[0]
    pargs = [x_res, pf, q] + gargs
    for n in ("pos1", "pos2", "att1", "att2", "lin_out"):
        pargs += [prm[n]["W"], _b(prm[n])]
    dfin = None
    if fin is not None:
        pargs += [fin["W"], _b(fin)]
        dfin = fin["W"].shape[1]
    (y,) = _ptb_post_fn(BN, d, K, dfin, BNtab, _pick_q(BN, K, BNtab))(*pargs)
    return y, x_res


def _run_td(xf_fine, pf_fine, np_l, idxf, prm, d, dout, BNf, BM, K=16):
    tab = jnp.concatenate(
        [xf_fine, pf_fine, jnp.zeros((BNf, 13), jnp.float32)], axis=1
    )
    use_sc = BNf >= _SC_MIN_TABLE_ROWS
    if use_sc:
        g = _gather_rows(tab, idxf).reshape(BM, K, d + 16)
        gargs, BNtab = [g], None
    else:
        gargs, BNtab = [idxf.reshape(BM * K, 1), tab], BNf
    W = prm["mlp"]["W"]
    (o,) = _td_fn(BM, d, dout, K, BNtab, _pick_q(BM, K, BNtab))(
        np_l.reshape(BM, 3), *gargs, W[:d], W[d:], _b(prm["mlp"])
    )
    return o


def _run_tu(xc_f, xs_f, dist, idxf, prm, dc, do, BMc, BMf):
    (x1,) = _linear_fn(BMc, dc, do)(xc_f, prm["lin1"]["W"], _b(prm["lin1"]))
    use_sc = BMc >= _SC_MIN_TABLE_ROWS
    if use_sc:
        g = _gather_rows(x1, idxf).reshape(BMf, 3, do)
        gargs, BNtab = [g], None
    else:
        gargs, BNtab = [idxf.reshape(BMf, 3), x1], BMc
    (y,) = _tu_fn(BMf, do, BNtab, _pick_q(BMf, 3, BNtab))(
        xs_f, dist.reshape(BMf, 3), *gargs, prm["lin2"]["W"], _b(prm["lin2"])
    )
    return y


def kernel(x, p, params):
    B, N, CIN = x.shape
    Ns = [N, N // 4, N // 16, N // 64, N // 256]
    ds = [32, 64, 128, 256, 512]
    Ks = [min(16, n) for n in Ns]

    xf0 = x.reshape(B * N, CIN)
    pf = [p.reshape(B * N, 3)]
    pT0 = jnp.transpose(p, (0, 2, 1))

    fo = _fps_fn(B, N)(pT0)
    npT = [pT0, fo[0], fo[2], fo[4], fo[6]]
    np3 = [p, fo[1], fo[3], fo[5], fo[7]]
    for l in range(1, 5):
        pf.append(np3[l].reshape(B * Ns[l], 3))

    iself = []
    for l in range(5):
        idx, _ = _knn(np3[l], npT[l], Ks[l])
        iself.append(idx.reshape(-1))

    y, x_skip0 = _run_ptb(
        xf0, pf[0], iself[0], params["PTB0"], ds[0], B * Ns[0], Ks[0],
        pre=params["MLP0"],
    )
    skips = {0: y}
    xcur = y

    enc_names = [("TD1", "PTB1e"), ("TD2", "PTB2e"), ("TD3", "PTB3e"), ("TD4", "PTB4e")]
    for l, (tdn, ptbn) in enumerate(enc_names, start=1):
        itd, _ = _knn(np3[l], npT[l - 1], 16)
        xcur = _run_td(
            xcur, pf[l - 1], np3[l], itd.reshape(-1), params[tdn],
            ds[l - 1], ds[l], B * Ns[l - 1], B * Ns[l],
        )
        xcur, _ = _run_ptb(
            xcur, pf[l], iself[l], params[ptbn], ds[l], B * Ns[l], Ks[l]
        )
        if l < 4:
            skips[l] = xcur

    xcur, _ = _run_ptb(
        xcur, pf[4], iself[4], params["PTBm"], ds[4], B * Ns[4], Ks[4],
        pre=params["MLP1"],
    )

    dec_names = [("TU1", "PTB1d"), ("TU2", "PTB2d"), ("TU3", "PTB3d"), ("TU4", "PTB4d")]
    for i, (tun, ptbn) in enumerate(dec_names):
        lc = 4 - i
        lf = lc - 1
        itu, dtu = _knn(np3[lf], npT[lc], 3)
        xcur = _run_tu(
            xcur, skips[lf], dtu, itu.reshape(-1), params[tun],
            ds[lc], ds[lf], B * Ns[lc], B * Ns[lf],
        )
        fin = params["out"] if ptbn == "PTB4d" else None
        xcur, _ = _run_ptb(
            xcur, pf[lf], iself[lf], params[ptbn], ds[lf], B * Ns[lf], Ks[lf],
            fin=fin,
        )

    ncls = params["out"]["W"].shape[1]
    return xcur.reshape(B, N, ncls), p


# speedup vs baseline: 13.5803x; 1.0270x over previous
import functools

import jax
import jax.numpy as jnp
from jax import lax
from jax.experimental import pallas as pl
from jax.experimental.pallas import tpu as pltpu
from jax.experimental.pallas import tpu_sc as plsc


_SC_INFO = None


def _sc_info():
    global _SC_INFO
    if _SC_INFO is None:
        _SC_INFO = plsc.get_sparse_core_info()
    return _SC_INFO


@functools.lru_cache(maxsize=None)
def _gather_fn(R, D, Bn):
    info = _sc_info()
    NC, NS = info.num_cores, info.num_subcores
    NW = NC * NS
    bpw = Bn // NW
    c = 8
    for cand in range(min(128, bpw), 7, -8):
        if bpw % cand == 0 and 8 * cand * D + 4 * bpw <= 440_000:
            c = cand
            break
    nch = bpw // c
    mesh = plsc.VectorSubcoreMesh(core_axis_name="c", subcore_axis_name="s")

    @functools.partial(
        pl.kernel,
        mesh=mesh,
        compiler_params=pltpu.CompilerParams(use_tc_tiling_on_sc=False),
        out_type=jax.ShapeDtypeStruct((Bn, D), jnp.float32),
        scratch_types=[
            pltpu.VMEM((bpw,), jnp.int32),
            pltpu.VMEM((c, D), jnp.float32),
            pltpu.VMEM((c, D), jnp.float32),
            pltpu.SemaphoreType.DMA,
            pltpu.SemaphoreType.DMA,
            pltpu.SemaphoreType.DMA,
            pltpu.SemaphoreType.DMA,
        ],
    )
    def k(table_hbm, idx_hbm, out_hbm, idx_all, r0, r1, g0, g1, w0, w1):
        wid = lax.axis_index("s") * NC + lax.axis_index("c")
        base = wid * bpw
        rows = (r0, r1)
        gsem = (g0, g1)
        wsem = (w0, w1)
        pltpu.sync_copy(idx_hbm.at[pl.ds(base, bpw)], idx_all)
        gcp = [None, None]
        wcp = [None, None]
        for j in range(nch):
            b = j & 1
            if j >= 2:
                wcp[b].wait()
            gcp[b] = pltpu.async_copy(
                table_hbm.at[idx_all.at[pl.ds(j * c, c)]], rows[b], gsem[b]
            )
            if j >= 1:
                pb = (j - 1) & 1
                gcp[pb].wait()
                wcp[pb] = pltpu.async_copy(
                    rows[pb], out_hbm.at[pl.ds(base + (j - 1) * c, c)], wsem[pb]
                )
        lb = (nch - 1) & 1
        gcp[lb].wait()
        wcp[lb] = pltpu.async_copy(
            rows[lb], out_hbm.at[pl.ds(base + (nch - 1) * c, c)], wsem[lb]
        )
        if nch >= 2:
            wcp[(nch - 2) & 1].wait()
        wcp[lb].wait()

    return k


def _gather_rows(table, idx):
    Bn0 = idx.shape[0]
    Bn = -(-Bn0 // 256) * 256
    if Bn != Bn0:
        idx = jnp.concatenate([idx, jnp.zeros((Bn - Bn0,), jnp.int32)])
    out = _gather_fn(table.shape[0], table.shape[1], Bn)(table, idx)
    return out[:Bn0]


@functools.lru_cache(maxsize=None)
def _knn_fn(B, Nq, Nk, K, Qb):
    def body(pq_ref, pkT_ref, idx_ref, dst_ref):
        b = pl.program_id(0)
        pq = pq_ref[0]
        pkT = pkT_ref[0]
        qx, qy, qz = pq[:, 0:1], pq[:, 1:2], pq[:, 2:3]
        kx, ky, kz = pkT[0:1, :], pkT[1:2, :], pkT[2:3, :]
        dx = qx - kx
        dy = qy - ky
        dz = qz - kz
        d = (dx * dx + dy * dy) + dz * dz
        iota = lax.broadcasted_iota(jnp.int32, (Qb, Nk), 1)
        cols_i, cols_d = [], []
        for j in range(K):
            m = jnp.min(d, axis=1, keepdims=True)
            am = jnp.min(jnp.where(d == m, iota, Nk), axis=1, keepdims=True)
            cols_i.append(am)
            cols_d.append(m)
            if j < K - 1:
                d = jnp.where(iota == am, jnp.float32(jnp.inf), d)
        idx_ref[0] = jnp.concatenate(cols_i, axis=1) + b * Nk
        dst_ref[0] = jnp.concatenate(cols_d, axis=1)

    grid = (B, Nq // Qb)
    return pl.pallas_call(
        body,
        grid=grid,
        in_specs=[
            pl.BlockSpec((1, Qb, 3), lambda b, i: (b, i, 0)),
            pl.BlockSpec((1, 3, Nk), lambda b, i: (b, 0, 0)),
        ],
        out_specs=[
            pl.BlockSpec((1, Qb, K), lambda b, i: (b, i, 0)),
            pl.BlockSpec((1, Qb, K), lambda b, i: (b, i, 0)),
        ],
        out_shape=[
            jax.ShapeDtypeStruct((B, Nq, K), jnp.int32),
            jax.ShapeDtypeStruct((B, Nq, K), jnp.float32),
        ],
    )


@functools.lru_cache(maxsize=None)
def _knn_small_fn(B, Nq, Nk, K):
    def body(pq_ref, pkT_ref, idx_ref, dst_ref):
        pq = pq_ref[...]
        pkT = pkT_ref[...]
        qx, qy, qz = pq[:, :, 0:1], pq[:, :, 1:2], pq[:, :, 2:3]
        kx = pkT[:, 0:1, :]
        ky = pkT[:, 1:2, :]
        kz = pkT[:, 2:3, :]
        dx = qx - kx
        dy = qy - ky
        dz = qz - kz
        d = (dx * dx + dy * dy) + dz * dz
        iota = lax.broadcasted_iota(jnp.int32, (B, Nq, Nk), 2)
        boff = lax.broadcasted_iota(jnp.int32, (B, Nq, 1), 0) * Nk
        cols_i, cols_d = [], []
        for j in range(K):
            m = jnp.min(d, axis=2, keepdims=True)
            am = jnp.min(jnp.where(d == m, iota, Nk), axis=2, keepdims=True)
            cols_i.append(am)
            cols_d.append(m)
            if j < K - 1:
                d = jnp.where(iota == am, jnp.float32(jnp.inf), d)
        idx_ref[...] = jnp.concatenate(cols_i, axis=2) + boff
        dst_ref[...] = jnp.concatenate(cols_d, axis=2)

    return pl.pallas_call(
        body,
        out_shape=[
            jax.ShapeDtypeStruct((B, Nq, K), jnp.int32),
            jax.ShapeDtypeStruct((B, Nq, K), jnp.float32),
        ],
    )


def _knn(pq, pkT, K):
    B, Nq, _ = pq.shape
    Nk = pkT.shape[2]
    if B * Nq * Nk * 4 <= 4 * 1024 * 1024:
        return _knn_small_fn(B, Nq, Nk, K)(pq, pkT)
    Qb = min(Nq, 512)
    return _knn_fn(B, Nq, Nk, K, Qb)(pq, pkT)


@functools.lru_cache(maxsize=None)
def _fps_fn(B, N):
    Ms = [N // 4, N // 16, N // 64, N // 256]

    def level(px, py, pz, M, npT_ref, np_ref):
        Ncur = px.shape[1]
        iota = lax.broadcasted_iota(jnp.int32, (B, Ncur), 1)
        iota3 = lax.broadcasted_iota(jnp.int32, (B, 3, M), 2)
        p0x, p0y, p0z = px[:, 0:1], py[:, 0:1], pz[:, 0:1]
        dx, dy, dz = px - p0x, py - p0y, pz - p0z
        mind0 = (dx * dx + dy * dy) + dz * dz
        cvec0 = jnp.concatenate(
            [p0x[:, None, :], p0y[:, None, :], p0z[:, None, :]], axis=1
        )
        tacc0 = cvec0 * (iota3 == 0).astype(jnp.float32)

        def step(i, carry):
            mind, tacc = carry
            m = jnp.max(mind, axis=1, keepdims=True)
            nxt = jnp.min(jnp.where(mind == m, iota, Ncur), axis=1, keepdims=True)
            sel = iota == nxt
            ptx = jnp.sum(jnp.where(sel, px, 0.0), axis=1, keepdims=True)
            pty = jnp.sum(jnp.where(sel, py, 0.0), axis=1, keepdims=True)
            ptz = jnp.sum(jnp.where(sel, pz, 0.0), axis=1, keepdims=True)
            ddx, ddy, ddz = px - ptx, py - pty, pz - ptz
            d = (ddx * ddx + ddy * ddy) + ddz * ddz
            cvec = jnp.concatenate(
                [ptx[:, None, :], pty[:, None, :], ptz[:, None, :]], axis=1
            )
            tacc = tacc + cvec * (iota3 == i).astype(jnp.float32)
            return jnp.minimum(mind, d), tacc

        _, tacc = lax.fori_loop(1, M, step, (mind0, tacc0))
        npT_ref[...] = tacc
        np_ref[...] = jnp.swapaxes(tacc, 1, 2)
        return tacc[:, 0, :], tacc[:, 1, :], tacc[:, 2, :]

    def body(pT_ref, o1T, o1, o2T, o2, o3T, o3, o4T, o4):
        pT = pT_ref[...]
        px, py, pz = pT[:, 0, :], pT[:, 1, :], pT[:, 2, :]
        px, py, pz = level(px, py, pz, Ms[0], o1T, o1)
        px, py, pz = level(px, py, pz, Ms[1], o2T, o2)
        px, py, pz = level(px, py, pz, Ms[2], o3T, o3)
        level(px, py, pz, Ms[3], o4T, o4)

    outs = []
    for M in Ms:
        outs.append(jax.ShapeDtypeStruct((B, 3, M), jnp.float32))
        outs.append(jax.ShapeDtypeStruct((B, M, 3), jnp.float32))
    return pl.pallas_call(body, out_shape=outs)


def _wspec(shape):
    n = len(shape)
    return pl.BlockSpec(shape, lambda i, _n=n: (0,) * _n)


@functools.lru_cache(maxsize=None)
def _ptb_pre_fn(BN, d, dpre):
    R = min(BN, 512)
    din = dpre if dpre is not None else d
    nw = 10 if dpre is not None else 8

    def body(x_ref, p_ref, *refs):
        ws = [r[...] for r in refs[:nw]]
        outs = refs[nw:]
        x = x_ref[...]
        i = 0
        if dpre is not None:
            x = x @ ws[0] + ws[1]
            i = 2
        Win, bin_, Wq, bq, Wk, bk, Wv, bv = ws[i : i + 8]
        h = x @ Win + bin_
        q = h @ Wq + bq
        kf = h @ Wk + bk
        vf = h @ Wv + bv
        tab = jnp.concatenate(
            [kf, vf, p_ref[...], jnp.zeros((R, 13), jnp.float32)], axis=1
        )
        outs[0][...] = q
        outs[1][...] = tab
        if dpre is not None:
            outs[2][...] = x

    def row(c):
        return pl.BlockSpec((R, c), lambda i: (i, 0))

    in_specs = [row(din), row(3)]
    if dpre is not None:
        in_specs += [_wspec((din, d)), _wspec((1, d))]
    in_specs += [_wspec((d, d)), _wspec((1, d))] * 4
    out_specs = [row(d), row(2 * d + 16)]
    out_shape = [
        jax.ShapeDtypeStruct((BN, d), jnp.float32),
        jax.ShapeDtypeStruct((BN, 2 * d + 16), jnp.float32),
    ]
    if dpre is not None:
        out_specs.append(row(d))
        out_shape.append(jax.ShapeDtypeStruct((BN, d), jnp.float32))
    return pl.pallas_call(
        body,
        grid=(BN // R,),
        in_specs=in_specs,
        out_specs=out_specs,
        out_shape=out_shape,
    )


@functools.lru_cache(maxsize=None)
def _ptb_post_fn(BN, d, K, dfin, BNtab, Q):
    nw = 12 if dfin is not None else 10
    dout = dfin if dfin is not None else d
    Dt = 2 * d + 16

    def body(x_ref, p_ref, q_ref, *refs):
        if BNtab is None:
            (g_ref,) = refs[:1]
            refs = refs[1:]
            g = g_ref[...]
        else:
            idx_ref, tab_ref = refs[:2]
            refs = refs[2:]
            iota = lax.broadcasted_iota(jnp.int32, (Q * K, BNtab), 1)
            onehot = (iota == idx_ref[...]).astype(jnp.float32)
            g = (onehot @ tab_ref[...]).reshape(Q, K, Dt)
        ws = [r[...] for r in refs[:nw]]
        y_ref = refs[nw]
        P1, b1, P2, b2, A1, a1, A2, a2, Wout, bout = ws[:10]
        kn = g[:, :, :d]
        vn = g[:, :, d : 2 * d]
        pg = g[:, :, 2 * d : 2 * d + 3]
        p3 = p_ref[...][:, None, :]
        rel = (p3 - pg).reshape(Q * K, 3)
        pos = jnp.maximum(rel @ P1 + b1, 0.0) @ P2 + b2
        qv = q_ref[...]
        t = (qv[:, None, :] - kn).reshape(Q * K, d) + pos
        a = (jnp.maximum(t @ A1 + a1, 0.0) @ A2 + a2).reshape(Q, K, d)
        m = jnp.max(a, axis=1, keepdims=True)
        e = jnp.exp(a - m)
        s = jnp.sum(e, axis=1, keepdims=True)
        a = e / s
        pos3 = pos.reshape(Q, K, d)
        o = jnp.sum(a * (vn + pos3), axis=1)
        y = x_ref[...] + o @ Wout + bout
        if dfin is not None:
            y = y @ ws[10] + ws[11]
        y_ref[...] = y

    def row(c, dt=None):
        return pl.BlockSpec((Q, c), lambda i: (i, 0))

    in_specs = [row(d), row(3), row(d)]
    if BNtab is None:
        in_specs += [pl.BlockSpec((Q, K, Dt), lambda i: (i, 0, 0))]
    else:
        in_specs += [
            pl.BlockSpec((Q * K, 1), lambda i: (i, 0)),
            _wspec((BNtab, Dt)),
        ]
    in_specs += [
        _wspec((3, d)), _wspec((1, d)),
        _wspec((d, d)), _wspec((1, d)),
        _wspec((d, d)), _wspec((1, d)),
        _wspec((d, d)), _wspec((1, d)),
        _wspec((d, d)), _wspec((1, d)),
    ]
    if dfin is not None:
        in_specs += [_wspec((d, dfin)), _wspec((1, dfin))]
    return pl.pallas_call(
        body,
        grid=(BN // Q,),
        in_specs=in_specs,
        out_specs=[row(dout)],
        out_shape=[jax.ShapeDtypeStruct((BN, dout), jnp.float32)],
    )


@functools.lru_cache(maxsize=None)
def _td_fn(BM, d, dout, K, BNtab, Q):
    Dt = d + 16

    def body(np_ref, *refs):
        if BNtab is None:
            (g_ref,) = refs[:1]
            refs = refs[1:]
            g = g_ref[...]
        else:
            idx_ref, tab_ref = refs[:2]
            refs = refs[2:]
            iota = lax.broadcasted_iota(jnp.int32, (Q * K, BNtab), 1)
            onehot = (iota == idx_ref[...]).astype(jnp.float32)
            g = (onehot @ tab_ref[...]).reshape(Q, K, Dt)
        w1_ref, w2_ref, b_ref, o_ref = refs
        nx = g[:, :, :d].reshape(Q * K, d)
        pg = g[:, :, d : d + 3]
        rel = (np_ref[...][:, None, :] - pg).reshape(Q * K, 3)
        feat = nx @ w1_ref[...] + rel @ w2_ref[...] + b_ref[...]
        feat = jnp.maximum(feat, 0.0).reshape(Q, K, dout)
        o_ref[...] = jnp.max(feat, axis=1)

    in_specs = [pl.BlockSpec((Q, 3), lambda i: (i, 0))]
    if BNtab is None:
        in_specs += [pl.BlockSpec((Q, K, Dt), lambda i: (i, 0, 0))]
    else:
        in_specs += [
            pl.BlockSpec((Q * K, 1), lambda i: (i, 0)),
            _wspec((BNtab, Dt)),
        ]
    in_specs += [_wspec((d, dout)), _wspec((3, dout)), _wspec((1, dout))]
    return pl.pallas_call(
        body,
        grid=(BM // Q,),
        in_specs=in_specs,
        out_specs=[pl.BlockSpec((Q, dout), lambda i: (i, 0))],
        out_shape=[jax.ShapeDtypeStruct((BM, dout), jnp.float32)],
    )


@functools.lru_cache(maxsize=None)
def _tu_fn(BMf, do, BNtab, Q):
    def body(xs_ref, dst_ref, *refs):
        w2_ref, b2_ref, o_ref = refs[-3:]
        x2 = xs_ref[...] @ w2_ref[...] + b2_ref[...]
        w = 1.0 / (dst_ref[...] + 1e-8)
        w = w / jnp.sum(w, axis=1, keepdims=True)
        if BNtab is None:
            g = refs[0][...]
            o = jnp.sum(g * w[:, :, None], axis=1)
        else:
            idx_ref, tab_ref = refs[:2]
            idx = idx_ref[...]
            tab = tab_ref[...]
            iota = lax.broadcasted_iota(jnp.int32, (Q, BNtab), 1)
            o = jnp.zeros((Q, do), jnp.float32)
            for j in range(3):
                oh = (iota == idx[:, j : j + 1]).astype(jnp.float32)
                o = o + (oh @ tab) * w[:, j : j + 1]
        o_ref[...] = x2 + o

    in_specs = [
        pl.BlockSpec((Q, do), lambda i: (i, 0)),
        pl.BlockSpec((Q, 3), lambda i: (i, 0)),
    ]
    if BNtab is None:
        in_specs += [pl.BlockSpec((Q, 3, do), lambda i: (i, 0, 0))]
    else:
        in_specs += [pl.BlockSpec((Q, 3), lambda i: (i, 0)), _wspec((BNtab, do))]
    in_specs += [_wspec((do, do)), _wspec((1, do))]
    return pl.pallas_call(
        body,
        grid=(BMf // Q,),
        in_specs=in_specs,
        out_specs=[pl.BlockSpec((Q, do), lambda i: (i, 0))],
        out_shape=[jax.ShapeDtypeStruct((BMf, do), jnp.float32)],
    )


@functools.lru_cache(maxsize=None)
def _linear_fn(BN, din, dout):
    R = min(BN, 512)

    def body(x_ref, w_ref, b_ref, o_ref):
        o_ref[...] = x_ref[...] @ w_ref[...] + b_ref[...]

    return pl.pallas_call(
        body,
        grid=(BN // R,),
        in_specs=[
            pl.BlockSpec((R, din), lambda i: (i, 0)),
            _wspec((din, dout)),
            _wspec((1, dout)),
        ],
        out_specs=[pl.BlockSpec((R, dout), lambda i: (i, 0))],
        out_shape=[jax.ShapeDtypeStruct((BN, dout), jnp.float32)],
    )


def _b(pr):
    return pr["b"].reshape(1, -1)


_SC_MIN_TABLE_ROWS = 2048


def _pick_q(BN, K, BNtab):
    q = min(BN, 512)
    if BNtab is not None:
        while q > 8 and q * K * BNtab * 4 > 4 * 1024 * 1024:
            q //= 2
    return q


def _run_ptb(xf, pf, idxf, prm, d, BN, K, pre=None, fin=None):
    args = [xf, pf]
    if pre is not None:
        args += [pre["W"], _b(pre)]
    for n in ("lin_in", "q", "k", "v"):
        args += [prm[n]["W"], _b(prm[n])]
    dpre = pre["W"].shape[0] if pre is not None else None
    outs = _ptb_pre_fn(BN, d, dpre)(*args)
    q, tab = outs[0], outs[1]
    x_res = outs[2] if pre is not None else xf
    Dt = 2 * d + 16
    use_sc = tab.shape[0] >= _SC_MIN_TABLE_ROWS
    if use_sc:
        g = _gather_rows(tab, idxf).reshape(BN, K, Dt)
        gargs, BNtab = [g], None
    else:
        gargs, BNtab = [idxf.reshape(BN * K, 1), tab], tab.shape[0]
    pargs = [x_res, pf, q] + gargs
    for n in ("pos1", "pos2", "att1", "att2", "lin_out"):
        pargs += [prm[n]["W"], _b(prm[n])]
    dfin = None
    if fin is not None:
        pargs += [fin["W"], _b(fin)]
        dfin = fin["W"].shape[1]
    (y,) = _ptb_post_fn(BN, d, K, dfin, BNtab, _pick_q(BN, K, BNtab))(*pargs)
    return y, x_res


def _run_td(xf_fine, pf_fine, np_l, idxf, prm, d, dout, BNf, BM, K=16):
    tab = jnp.concatenate(
        [xf_fine, pf_fine, jnp.zeros((BNf, 13), jnp.float32)], axis=1
    )
    use_sc = BNf >= _SC_MIN_TABLE_ROWS
    if use_sc:
        g = _gather_rows(tab, idxf).reshape(BM, K, d + 16)
        gargs, BNtab = [g], None
    else:
        gargs, BNtab = [idxf.reshape(BM * K, 1), tab], BNf
    W = prm["mlp"]["W"]
    (o,) = _td_fn(BM, d, dout, K, BNtab, _pick_q(BM, K, BNtab))(
        np_l.reshape(BM, 3), *gargs, W[:d], W[d:], _b(prm["mlp"])
    )
    return o


def _run_tu(xc_f, xs_f, dist, idxf, prm, dc, do, BMc, BMf):
    (x1,) = _linear_fn(BMc, dc, do)(xc_f, prm["lin1"]["W"], _b(prm["lin1"]))
    use_sc = BMc >= _SC_MIN_TABLE_ROWS
    if use_sc:
        g = _gather_rows(x1, idxf).reshape(BMf, 3, do)
        gargs, BNtab = [g], None
    else:
        gargs, BNtab = [idxf.reshape(BMf, 3), x1], BMc
    (y,) = _tu_fn(BMf, do, BNtab, _pick_q(BMf, 3, BNtab))(
        xs_f, dist.reshape(BMf, 3), *gargs, prm["lin2"]["W"], _b(prm["lin2"])
    )
    return y


def kernel(x, p, params):
    B, N, CIN = x.shape
    Ns = [N, N // 4, N // 16, N // 64, N // 256]
    ds = [32, 64, 128, 256, 512]
    Ks = [min(16, n) for n in Ns]

    xf0 = x.reshape(B * N, CIN)
    pf = [p.reshape(B * N, 3)]
    pT0 = jnp.transpose(p, (0, 2, 1))

    fo = _fps_fn(B, N)(pT0)
    npT = [pT0, fo[0], fo[2], fo[4], fo[6]]
    np3 = [p, fo[1], fo[3], fo[5], fo[7]]
    for l in range(1, 5):
        pf.append(np3[l].reshape(B * Ns[l], 3))

    iself = []
    for l in range(5):
        idx, _ = _knn(np3[l], npT[l], Ks[l])
        iself.append(idx.reshape(-1))

    y, x_skip0 = _run_ptb(
        xf0, pf[0], iself[0], params["PTB0"], ds[0], B * Ns[0], Ks[0],
        pre=params["MLP0"],
    )
    skips = {0: y}
    xcur = y

    enc_names = [("TD1", "PTB1e"), ("TD2", "PTB2e"), ("TD3", "PTB3e"), ("TD4", "PTB4e")]
    for l, (tdn, ptbn) in enumerate(enc_names, start=1):
        itd, _ = _knn(np3[l], npT[l - 1], 16)
        xcur = _run_td(
            xcur, pf[l - 1], np3[l], itd.reshape(-1), params[tdn],
            ds[l - 1], ds[l], B * Ns[l - 1], B * Ns[l],
        )
        xcur, _ = _run_ptb(
            xcur, pf[l], iself[l], params[ptbn], ds[l], B * Ns[l], Ks[l]
        )
        if l < 4:
            skips[l] = xcur

    xcur, _ = _run_ptb(
        xcur, pf[4], iself[4], params["PTBm"], ds[4], B * Ns[4], Ks[4],
        pre=params["MLP1"],
    )

    dec_names = [("TU1", "PTB1d"), ("TU2", "PTB2d"), ("TU3", "PTB3d"), ("TU4", "PTB4d")]
    for i, (tun, ptbn) in enumerate(dec_names):
        lc = 4 - i
        lf = lc - 1
        itu, dtu = _knn(np3[lf], npT[lc], 3)
        xcur = _run_tu(
            xcur, skips[lf], dtu, itu.reshape(-1), params[tun],
            ds[lc], ds[lf], B * Ns[lc], B * Ns[lf],
        )
        fin = params["out"] if ptbn == "PTB4d" else None
        xcur, _ = _run_ptb(
            xcur, pf[lf], iself[lf], params[ptbn], ds[lf], B * Ns[lf], Ks[lf],
            fin=fin,
        )

    ncls = params["out"]["W"].shape[1]
    return xcur.reshape(B, N, ncls), p


# speedup vs baseline: 13.6432x; 1.0046x over previous
import functools

import jax
import jax.numpy as jnp
from jax import lax
from jax.experimental import pallas as pl
from jax.experimental.pallas import tpu as pltpu
from jax.experimental.pallas import tpu_sc as plsc


_SC_INFO = None


def _sc_info():
    global _SC_INFO
    if _SC_INFO is None:
        _SC_INFO = plsc.get_sparse_core_info()
    return _SC_INFO


@functools.lru_cache(maxsize=None)
def _gather_fn(R, D, Bn):
    info = _sc_info()
    NC, NS = info.num_cores, info.num_subcores
    NW = NC * NS
    bpw = Bn // NW
    c = 8
    for cand in range(min(128, bpw), 7, -8):
        if bpw % cand == 0 and 8 * cand * D + 4 * bpw <= 440_000:
            c = cand
            break
    nch = bpw // c
    mesh = plsc.VectorSubcoreMesh(core_axis_name="c", subcore_axis_name="s")

    @functools.partial(
        pl.kernel,
        mesh=mesh,
        compiler_params=pltpu.CompilerParams(use_tc_tiling_on_sc=False),
        out_type=jax.ShapeDtypeStruct((Bn, D), jnp.float32),
        scratch_types=[
            pltpu.VMEM((bpw,), jnp.int32),
            pltpu.VMEM((c, D), jnp.float32),
            pltpu.VMEM((c, D), jnp.float32),
            pltpu.SemaphoreType.DMA,
            pltpu.SemaphoreType.DMA,
            pltpu.SemaphoreType.DMA,
            pltpu.SemaphoreType.DMA,
        ],
    )
    def k(table_hbm, idx_hbm, out_hbm, idx_all, r0, r1, g0, g1, w0, w1):
        wid = lax.axis_index("s") * NC + lax.axis_index("c")
        base = wid * bpw
        rows = (r0, r1)
        gsem = (g0, g1)
        wsem = (w0, w1)
        pltpu.sync_copy(idx_hbm.at[pl.ds(base, bpw)], idx_all)
        gcp = [None, None]
        wcp = [None, None]
        for j in range(nch):
            b = j & 1
            if j >= 2:
                wcp[b].wait()
            gcp[b] = pltpu.async_copy(
                table_hbm.at[idx_all.at[pl.ds(j * c, c)]], rows[b], gsem[b]
            )
            if j >= 1:
                pb = (j - 1) & 1
                gcp[pb].wait()
                wcp[pb] = pltpu.async_copy(
                    rows[pb], out_hbm.at[pl.ds(base + (j - 1) * c, c)], wsem[pb]
                )
        lb = (nch - 1) & 1
        gcp[lb].wait()
        wcp[lb] = pltpu.async_copy(
            rows[lb], out_hbm.at[pl.ds(base + (nch - 1) * c, c)], wsem[lb]
        )
        if nch >= 2:
            wcp[(nch - 2) & 1].wait()
        wcp[lb].wait()

    return k


def _gather_rows(table, idx):
    Bn0 = idx.shape[0]
    Bn = -(-Bn0 // 256) * 256
    if Bn != Bn0:
        idx = jnp.concatenate([idx, jnp.zeros((Bn - Bn0,), jnp.int32)])
    out = _gather_fn(table.shape[0], table.shape[1], Bn)(table, idx)
    return out[:Bn0]


@functools.lru_cache(maxsize=None)
def _knn_fn(B, Nq, Nk, K, Qb):
    def body(pq_ref, pkT_ref, idx_ref, dst_ref):
        b = pl.program_id(0)
        pq = pq_ref[0]
        pkT = pkT_ref[0]
        qx, qy, qz = pq[:, 0:1], pq[:, 1:2], pq[:, 2:3]
        kx, ky, kz = pkT[0:1, :], pkT[1:2, :], pkT[2:3, :]
        dx = qx - kx
        dy = qy - ky
        dz = qz - kz
        d = (dx * dx + dy * dy) + dz * dz
        iota = lax.broadcasted_iota(jnp.int32, (Qb, Nk), 1)
        cols_i, cols_d = [], []
        for j in range(K):
            m = jnp.min(d, axis=1, keepdims=True)
            am = jnp.min(jnp.where(d == m, iota, Nk), axis=1, keepdims=True)
            cols_i.append(am)
            cols_d.append(m)
            if j < K - 1:
                d = jnp.where(iota == am, jnp.float32(jnp.inf), d)
        idx_ref[0] = jnp.concatenate(cols_i, axis=1) + b * Nk
        dst_ref[0] = jnp.concatenate(cols_d, axis=1)

    grid = (B, Nq // Qb)
    return pl.pallas_call(
        body,
        grid=grid,
        in_specs=[
            pl.BlockSpec((1, Qb, 3), lambda b, i: (b, i, 0)),
            pl.BlockSpec((1, 3, Nk), lambda b, i: (b, 0, 0)),
        ],
        out_specs=[
            pl.BlockSpec((1, Qb, K), lambda b, i: (b, i, 0)),
            pl.BlockSpec((1, Qb, K), lambda b, i: (b, i, 0)),
        ],
        out_shape=[
            jax.ShapeDtypeStruct((B, Nq, K), jnp.int32),
            jax.ShapeDtypeStruct((B, Nq, K), jnp.float32),
        ],
    )


@functools.lru_cache(maxsize=None)
def _knn_small_fn(B, Nq, Nk, K):
    def body(pq_ref, pkT_ref, idx_ref, dst_ref):
        pq = pq_ref[...]
        pkT = pkT_ref[...]
        qx, qy, qz = pq[:, :, 0:1], pq[:, :, 1:2], pq[:, :, 2:3]
        kx = pkT[:, 0:1, :]
        ky = pkT[:, 1:2, :]
        kz = pkT[:, 2:3, :]
        dx = qx - kx
        dy = qy - ky
        dz = qz - kz
        d = (dx * dx + dy * dy) + dz * dz
        iota = lax.broadcasted_iota(jnp.int32, (B, Nq, Nk), 2)
        boff = lax.broadcasted_iota(jnp.int32, (B, Nq, 1), 0) * Nk
        cols_i, cols_d = [], []
        for j in range(K):
            m = jnp.min(d, axis=2, keepdims=True)
            am = jnp.min(jnp.where(d == m, iota, Nk), axis=2, keepdims=True)
            cols_i.append(am)
            cols_d.append(m)
            if j < K - 1:
                d = jnp.where(iota == am, jnp.float32(jnp.inf), d)
        idx_ref[...] = jnp.concatenate(cols_i, axis=2) + boff
        dst_ref[...] = jnp.concatenate(cols_d, axis=2)

    return pl.pallas_call(
        body,
        out_shape=[
            jax.ShapeDtypeStruct((B, Nq, K), jnp.int32),
            jax.ShapeDtypeStruct((B, Nq, K), jnp.float32),
        ],
    )


def _knn(pq, pkT, K):
    B, Nq, _ = pq.shape
    Nk = pkT.shape[2]
    if B * Nq * Nk * 4 <= 4 * 1024 * 1024:
        return _knn_small_fn(B, Nq, Nk, K)(pq, pkT)
    Qb = min(Nq, 512)
    return _knn_fn(B, Nq, Nk, K, Qb)(pq, pkT)


@functools.lru_cache(maxsize=None)
def _fps_fn(B, N):
    Ms = [N // 4, N // 16, N // 64, N // 256]

    def level(px, py, pz, M, npT_ref, np_ref):
        Ncur = px.shape[1]
        iota = lax.broadcasted_iota(jnp.int32, (B, Ncur), 1)
        iota3 = lax.broadcasted_iota(jnp.int32, (B, 3, M), 2)
        p0x, p0y, p0z = px[:, 0:1], py[:, 0:1], pz[:, 0:1]
        dx, dy, dz = px - p0x, py - p0y, pz - p0z
        mind0 = (dx * dx + dy * dy) + dz * dz
        cvec0 = jnp.concatenate(
            [p0x[:, None, :], p0y[:, None, :], p0z[:, None, :]], axis=1
        )
        tacc0 = cvec0 * (iota3 == 0).astype(jnp.float32)

        def step(i, carry):
            mind, tacc = carry
            m = jnp.max(mind, axis=1, keepdims=True)
            nxt = jnp.min(jnp.where(mind == m, iota, Ncur), axis=1, keepdims=True)
            sel = iota == nxt
            ptx = jnp.sum(jnp.where(sel, px, 0.0), axis=1, keepdims=True)
            pty = jnp.sum(jnp.where(sel, py, 0.0), axis=1, keepdims=True)
            ptz = jnp.sum(jnp.where(sel, pz, 0.0), axis=1, keepdims=True)
            ddx, ddy, ddz = px - ptx, py - pty, pz - ptz
            d = (ddx * ddx + ddy * ddy) + ddz * ddz
            cvec = jnp.concatenate(
                [ptx[:, None, :], pty[:, None, :], ptz[:, None, :]], axis=1
            )
            tacc = tacc + cvec * (iota3 == i).astype(jnp.float32)
            return jnp.minimum(mind, d), tacc

        _, tacc = lax.fori_loop(1, M, step, (mind0, tacc0))
        npT_ref[...] = tacc
        np_ref[...] = jnp.swapaxes(tacc, 1, 2)
        return tacc[:, 0, :], tacc[:, 1, :], tacc[:, 2, :]

    def body(pT_ref, o1T, o1, o2T, o2, o3T, o3, o4T, o4):
        pT = pT_ref[...]
        px, py, pz = pT[:, 0, :], pT[:, 1, :], pT[:, 2, :]
        px, py, pz = level(px, py, pz, Ms[0], o1T, o1)
        px, py, pz = level(px, py, pz, Ms[1], o2T, o2)
        px, py, pz = level(px, py, pz, Ms[2], o3T, o3)
        level(px, py, pz, Ms[3], o4T, o4)

    outs = []
    for M in Ms:
        outs.append(jax.ShapeDtypeStruct((B, 3, M), jnp.float32))
        outs.append(jax.ShapeDtypeStruct((B, M, 3), jnp.float32))
    return pl.pallas_call(body, out_shape=outs)


def _wspec(shape):
    n = len(shape)
    return pl.BlockSpec(shape, lambda i, _n=n: (0,) * _n)


@functools.lru_cache(maxsize=None)
def _ptb_pre_fn(BN, d, dpre):
    R = min(BN, 512)
    din = dpre if dpre is not None else d
    nw = 10 if dpre is not None else 8

    def body(x_ref, p_ref, *refs):
        ws = [r[...] for r in refs[:nw]]
        outs = refs[nw:]
        x = x_ref[...]
        i = 0
        if dpre is not None:
            x = x @ ws[0] + ws[1]
            i = 2
        Win, bin_, Wq, bq, Wk, bk, Wv, bv = ws[i : i + 8]
        h = x @ Win + bin_
        q = h @ Wq + bq
        kf = h @ Wk + bk
        vf = h @ Wv + bv
        tab = jnp.concatenate(
            [kf, vf, p_ref[...], jnp.zeros((R, 13), jnp.float32)], axis=1
        )
        outs[0][...] = q
        outs[1][...] = tab
        if dpre is not None:
            outs[2][...] = x

    def row(c):
        return pl.BlockSpec((R, c), lambda i: (i, 0))

    in_specs = [row(din), row(3)]
    if dpre is not None:
        in_specs += [_wspec((din, d)), _wspec((1, d))]
    in_specs += [_wspec((d, d)), _wspec((1, d))] * 4
    out_specs = [row(d), row(2 * d + 16)]
    out_shape = [
        jax.ShapeDtypeStruct((BN, d), jnp.float32),
        jax.ShapeDtypeStruct((BN, 2 * d + 16), jnp.float32),
    ]
    if dpre is not None:
        out_specs.append(row(d))
        out_shape.append(jax.ShapeDtypeStruct((BN, d), jnp.float32))
    return pl.pallas_call(
        body,
        grid=(BN // R,),
        in_specs=in_specs,
        out_specs=out_specs,
        out_shape=out_shape,
    )


@functools.lru_cache(maxsize=None)
def _ptb_post_fn(BN, d, K, dfin, BNtab, Q):
    nw = 12 if dfin is not None else 10
    dout = dfin if dfin is not None else d
    Dt = 2 * d + 16

    def body(x_ref, p_ref, q_ref, *refs):
        if BNtab is None:
            (g_ref,) = refs[:1]
            refs = refs[1:]
            g = g_ref[...]
        else:
            idx_ref, tab_ref = refs[:2]
            refs = refs[2:]
            iota = lax.broadcasted_iota(jnp.int32, (Q * K, BNtab), 1)
            onehot = (iota == idx_ref[...]).astype(jnp.float32)
            g = (onehot @ tab_ref[...]).reshape(Q, K, Dt)
        ws = [r[...] for r in refs[:nw]]
        y_ref = refs[nw]
        P1, b1, P2, b2, A1, a1, A2, a2, Wout, bout = ws[:10]
        kn = g[:, :, :d]
        vn = g[:, :, d : 2 * d]
        pg = g[:, :, 2 * d : 2 * d + 3]
        p3 = p_ref[...][:, None, :]
        rel = (p3 - pg).reshape(Q * K, 3)
        pos = jnp.maximum(rel @ P1 + b1, 0.0) @ P2 + b2
        qv = q_ref[...]
        t = (qv[:, None, :] - kn).reshape(Q * K, d) + pos
        a = (jnp.maximum(t @ A1 + a1, 0.0) @ A2 + a2).reshape(Q, K, d)
        m = jnp.max(a, axis=1, keepdims=True)
        e = jnp.exp(a - m)
        s = jnp.sum(e, axis=1, keepdims=True)
        a = e / s
        pos3 = pos.reshape(Q, K, d)
        o = jnp.sum(a * (vn + pos3), axis=1)
        y = x_ref[...] + o @ Wout + bout
        if dfin is not None:
            y = y @ ws[10] + ws[11]
        y_ref[...] = y

    def row(c, dt=None):
        return pl.BlockSpec((Q, c), lambda i: (i, 0))

    in_specs = [row(d), row(3), row(d)]
    if BNtab is None:
        in_specs += [pl.BlockSpec((Q, K, Dt), lambda i: (i, 0, 0))]
    else:
        in_specs += [
            pl.BlockSpec((Q * K, 1), lambda i: (i, 0)),
            _wspec((BNtab, Dt)),
        ]
    in_specs += [
        _wspec((3, d)), _wspec((1, d)),
        _wspec((d, d)), _wspec((1, d)),
        _wspec((d, d)), _wspec((1, d)),
        _wspec((d, d)), _wspec((1, d)),
        _wspec((d, d)), _wspec((1, d)),
    ]
    if dfin is not None:
        in_specs += [_wspec((d, dfin)), _wspec((1, dfin))]
    return pl.pallas_call(
        body,
        grid=(BN // Q,),
        in_specs=in_specs,
        out_specs=[row(dout)],
        out_shape=[jax.ShapeDtypeStruct((BN, dout), jnp.float32)],
    )


@functools.lru_cache(maxsize=None)
def _ptb_fused_fn(BN, d, K, dpre, dfin, Q):
    nw = 18 + (2 if dpre is not None else 0) + (2 if dfin is not None else 0)
    din = dpre if dpre is not None else d
    dout = dfin if dfin is not None else d
    Dt = 2 * d + 16

    def body(xb_ref, pb_ref, xfull_ref, pfull_ref, idx_ref, *refs):
        ws = [r[...] for r in refs[:nw]]
        y_ref = refs[nw]
        i = 0
        if dpre is not None:
            Wpre, bpre = ws[0], ws[1]
            i = 2
        (Win, bin_, Wq, bq, Wk, bk, Wv, bv, P1, b1, P2, b2,
         A1, a1, A2, a2, Wout, bout) = ws[i : i + 18]
        xfull = xfull_ref[...]
        xb = xb_ref[...]
        if dpre is not None:
            xfull = xfull @ Wpre + bpre
            xb = xb @ Wpre + bpre
        hf = xfull @ Win + bin_
        kf = hf @ Wk + bk
        vf = hf @ Wv + bv
        tab = jnp.concatenate(
            [kf, vf, pfull_ref[...], jnp.zeros((BN, 13), jnp.float32)], axis=1
        )
        hb = xb @ Win + bin_
        qv = hb @ Wq + bq
        iota = lax.broadcasted_iota(jnp.int32, (Q * K, BN), 1)
        onehot = (iota == idx_ref[...]).astype(jnp.float32)
        g = (onehot @ tab).reshape(Q, K, Dt)
        kn = g[:, :, :d]
        vn = g[:, :, d : 2 * d]
        pg = g[:, :, 2 * d : 2 * d + 3]
        p3 = pb_ref[...][:, None, :]
        rel = (p3 - pg).reshape(Q * K, 3)
        pos = jnp.maximum(rel @ P1 + b1, 0.0) @ P2 + b2
        t = (qv[:, None, :] - kn).reshape(Q * K, d) + pos
        a = (jnp.maximum(t @ A1 + a1, 0.0) @ A2 + a2).reshape(Q, K, d)
        m = jnp.max(a, axis=1, keepdims=True)
        e = jnp.exp(a - m)
        s = jnp.sum(e, axis=1, keepdims=True)
        a = e / s
        pos3 = pos.reshape(Q, K, d)
        o = jnp.sum(a * (vn + pos3), axis=1)
        y = xb + o @ Wout + bout
        if dfin is not None:
            y = y @ ws[nw - 2] + ws[nw - 1]
        y_ref[...] = y

    def row(c):
        return pl.BlockSpec((Q, c), lambda i: (i, 0))

    in_specs = [
        row(din),
        row(3),
        _wspec((BN, din)),
        _wspec((BN, 3)),
        pl.BlockSpec((Q * K, 1), lambda i: (i, 0)),
    ]
    if dpre is not None:
        in_specs += [_wspec((din, d)), _wspec((1, d))]
    in_specs += [_wspec((d, d)), _wspec((1, d))] * 4
    in_specs += [_wspec((3, d)), _wspec((1, d))]
    in_specs += [_wspec((d, d)), _wspec((1, d))] * 4
    if dfin is not None:
        in_specs += [_wspec((d, dfin)), _wspec((1, dfin))]
    return pl.pallas_call(
        body,
        grid=(BN // Q,),
        in_specs=in_specs,
        out_specs=[row(dout)],
        out_shape=[jax.ShapeDtypeStruct((BN, dout), jnp.float32)],
    )


@functools.lru_cache(maxsize=None)
def _td_fn(BM, d, dout, K, BNtab, Q):
    Dt = d + 16

    def body(np_ref, *refs):
        if BNtab is None:
            (g_ref,) = refs[:1]
            refs = refs[1:]
            g = g_ref[...]
        else:
            idx_ref, tab_ref = refs[:2]
            refs = refs[2:]
            iota = lax.broadcasted_iota(jnp.int32, (Q * K, BNtab), 1)
            onehot = (iota == idx_ref[...]).astype(jnp.float32)
            g = (onehot @ tab_ref[...]).reshape(Q, K, Dt)
        w1_ref, w2_ref, b_ref, o_ref = refs
        nx = g[:, :, :d].reshape(Q * K, d)
        pg = g[:, :, d : d + 3]
        rel = (np_ref[...][:, None, :] - pg).reshape(Q * K, 3)
        feat = nx @ w1_ref[...] + rel @ w2_ref[...] + b_ref[...]
        feat = jnp.maximum(feat, 0.0).reshape(Q, K, dout)
        o_ref[...] = jnp.max(feat, axis=1)

    in_specs = [pl.BlockSpec((Q, 3), lambda i: (i, 0))]
    if BNtab is None:
        in_specs += [pl.BlockSpec((Q, K, Dt), lambda i: (i, 0, 0))]
    else:
        in_specs += [
            pl.BlockSpec((Q * K, 1), lambda i: (i, 0)),
            _wspec((BNtab, Dt)),
        ]
    in_specs += [_wspec((d, dout)), _wspec((3, dout)), _wspec((1, dout))]
    return pl.pallas_call(
        body,
        grid=(BM // Q,),
        in_specs=in_specs,
        out_specs=[pl.BlockSpec((Q, dout), lambda i: (i, 0))],
        out_shape=[jax.ShapeDtypeStruct((BM, dout), jnp.float32)],
    )


@functools.lru_cache(maxsize=None)
def _tu_fn(BMf, do, BNtab, Q):
    def body(xs_ref, dst_ref, *refs):
        w2_ref, b2_ref, o_ref = refs[-3:]
        x2 = xs_ref[...] @ w2_ref[...] + b2_ref[...]
        w = 1.0 / (dst_ref[...] + 1e-8)
        w = w / jnp.sum(w, axis=1, keepdims=True)
        if BNtab is None:
            g = refs[0][...]
            o = jnp.sum(g * w[:, :, None], axis=1)
        else:
            idx_ref, tab_ref = refs[:2]
            idx = idx_ref[...]
            tab = tab_ref[...]
            iota = lax.broadcasted_iota(jnp.int32, (Q, BNtab), 1)
            o = jnp.zeros((Q, do), jnp.float32)
            for j in range(3):
                oh = (iota == idx[:, j : j + 1]).astype(jnp.float32)
                o = o + (oh @ tab) * w[:, j : j + 1]
        o_ref[...] = x2 + o

    in_specs = [
        pl.BlockSpec((Q, do), lambda i: (i, 0)),
        pl.BlockSpec((Q, 3), lambda i: (i, 0)),
    ]
    if BNtab is None:
        in_specs += [pl.BlockSpec((Q, 3, do), lambda i: (i, 0, 0))]
    else:
        in_specs += [pl.BlockSpec((Q, 3), lambda i: (i, 0)), _wspec((BNtab, do))]
    in_specs += [_wspec((do, do)), _wspec((1, do))]
    return pl.pallas_call(
        body,
        grid=(BMf // Q,),
        in_specs=in_specs,
        out_specs=[pl.BlockSpec((Q, do), lambda i: (i, 0))],
        out_shape=[jax.ShapeDtypeStruct((BMf, do), jnp.float32)],
    )


@functools.lru_cache(maxsize=None)
def _linear_fn(BN, din, dout):
    R = min(BN, 512)

    def body(x_ref, w_ref, b_ref, o_ref):
        o_ref[...] = x_ref[...] @ w_ref[...] + b_ref[...]

    return pl.pallas_call(
        body,
        grid=(BN // R,),
        in_specs=[
            pl.BlockSpec((R, din), lambda i: (i, 0)),
            _wspec((din, dout)),
            _wspec((1, dout)),
        ],
        out_specs=[pl.BlockSpec((R, dout), lambda i: (i, 0))],
        out_shape=[jax.ShapeDtypeStruct((BN, dout), jnp.float32)],
    )


def _b(pr):
    return pr["b"].reshape(1, -1)


_SC_MIN_TABLE_ROWS = 2048


def _pick_q(BN, K, BNtab):
    q = min(BN, 512)
    if BNtab is not None:
        while q > 8 and q * K * BNtab * 4 > 4 * 1024 * 1024:
            q //= 2
    return q


def _run_ptb(xf, pf, idxf, prm, d, BN, K, pre=None, fin=None):
    dpre0 = pre["W"].shape[0] if pre is not None else None
    dfin0 = fin["W"].shape[1] if fin is not None else None
    if BN < _SC_MIN_TABLE_ROWS:
        Q = _pick_q(BN, K, BN)
        fargs = [xf, pf, xf, pf, idxf.reshape(BN * K, 1)]
        if pre is not None:
            fargs += [pre["W"], _b(pre)]
        for n in ("lin_in", "q", "k", "v", "pos1", "pos2", "att1", "att2",
                  "lin_out"):
            fargs += [prm[n]["W"], _b(prm[n])]
        if fin is not None:
            fargs += [fin["W"], _b(fin)]
        (y,) = _ptb_fused_fn(BN, d, K, dpre0, dfin0, Q)(*fargs)
        return y, xf
    args = [xf, pf]
    if pre is not None:
        args += [pre["W"], _b(pre)]
    for n in ("lin_in", "q", "k", "v"):
        args += [prm[n]["W"], _b(prm[n])]
    dpre = pre["W"].shape[0] if pre is not None else None
    outs = _ptb_pre_fn(BN, d, dpre)(*args)
    q, tab = outs[0], outs[1]
    x_res = outs[2] if pre is not None else xf
    Dt = 2 * d + 16
    use_sc = tab.shape[0] >= _SC_MIN_TABLE_ROWS
    if use_sc:
        g = _gather_rows(tab, idxf).reshape(BN, K, Dt)
        gargs, BNtab = [g], None
    else:
        gargs, BNtab = [idxf.reshape(BN * K, 1), tab], tab.shape[0]
    pargs = [x_res, pf, q] + gargs
    for n in ("pos1", "pos2", "att1", "att2", "lin_out"):
        pargs += [prm[n]["W"], _b(prm[n])]
    dfin = None
    if fin is not None:
        pargs += [fin["W"], _b(fin)]
        dfin = fin["W"].shape[1]
    (y,) = _ptb_post_fn(BN, d, K, dfin, BNtab, _pick_q(BN, K, BNtab))(*pargs)
    return y, x_res


def _run_td(xf_fine, pf_fine, np_l, idxf, prm, d, dout, BNf, BM, K=16):
    tab = jnp.concatenate(
        [xf_fine, pf_fine, jnp.zeros((BNf, 13), jnp.float32)], axis=1
    )
    use_sc = BNf >= _SC_MIN_TABLE_ROWS
    if use_sc:
        g = _gather_rows(tab, idxf).reshape(BM, K, d + 16)
        gargs, BNtab = [g], None
    else:
        gargs, BNtab = [idxf.reshape(BM * K, 1), tab], BNf
    W = prm["mlp"]["W"]
    (o,) = _td_fn(BM, d, dout, K, BNtab, _pick_q(BM, K, BNtab))(
        np_l.reshape(BM, 3), *gargs, W[:d], W[d:], _b(prm["mlp"])
    )
    return o


def _run_tu(xc_f, xs_f, dist, idxf, prm, dc, do, BMc, BMf):
    (x1,) = _linear_fn(BMc, dc, do)(xc_f, prm["lin1"]["W"], _b(prm["lin1"]))
    use_sc = BMc >= _SC_MIN_TABLE_ROWS
    if use_sc:
        g = _gather_rows(x1, idxf).reshape(BMf, 3, do)
        gargs, BNtab = [g], None
    else:
        gargs, BNtab = [idxf.reshape(BMf, 3), x1], BMc
    (y,) = _tu_fn(BMf, do, BNtab, _pick_q(BMf, 3, BNtab))(
        xs_f, dist.reshape(BMf, 3), *gargs, prm["lin2"]["W"], _b(prm["lin2"])
    )
    return y


def kernel(x, p, params):
    B, N, CIN = x.shape
    Ns = [N, N // 4, N // 16, N // 64, N // 256]
    ds = [32, 64, 128, 256, 512]
    Ks = [min(16, n) for n in Ns]

    xf0 = x.reshape(B * N, CIN)
    pf = [p.reshape(B * N, 3)]
    pT0 = jnp.transpose(p, (0, 2, 1))

    fo = _fps_fn(B, N)(pT0)
    npT = [pT0, fo[0], fo[2], fo[4], fo[6]]
    np3 = [p, fo[1], fo[3], fo[5], fo[7]]
    for l in range(1, 5):
        pf.append(np3[l].reshape(B * Ns[l], 3))

    iself = []
    for l in range(5):
        idx, _ = _knn(np3[l], npT[l], Ks[l])
        iself.append(idx.reshape(-1))

    y, x_skip0 = _run_ptb(
        xf0, pf[0], iself[0], params["PTB0"], ds[0], B * Ns[0], Ks[0],
        pre=params["MLP0"],
    )
    skips = {0: y}
    xcur = y

    enc_names = [("TD1", "PTB1e"), ("TD2", "PTB2e"), ("TD3", "PTB3e"), ("TD4", "PTB4e")]
    for l, (tdn, ptbn) in enumerate(enc_names, start=1):
        itd, _ = _knn(np3[l], npT[l - 1], 16)
        xcur = _run_td(
            xcur, pf[l - 1], np3[l], itd.reshape(-1), params[tdn],
            ds[l - 1], ds[l], B * Ns[l - 1], B * Ns[l],
        )
        xcur, _ = _run_ptb(
            xcur, pf[l], iself[l], params[ptbn], ds[l], B * Ns[l], Ks[l]
        )
        if l < 4:
            skips[l] = xcur

    xcur, _ = _run_ptb(
        xcur, pf[4], iself[4], params["PTBm"], ds[4], B * Ns[4], Ks[4],
        pre=params["MLP1"],
    )

    dec_names = [("TU1", "PTB1d"), ("TU2", "PTB2d"), ("TU3", "PTB3d"), ("TU4", "PTB4d")]
    for i, (tun, ptbn) in enumerate(dec_names):
        lc = 4 - i
        lf = lc - 1
        itu, dtu = _knn(np3[lf], npT[lc], 3)
        xcur = _run_tu(
            xcur, skips[lf], dtu, itu.reshape(-1), params[tun],
            ds[lc], ds[lf], B * Ns[lc], B * Ns[lf],
        )
        fin = params["out"] if ptbn == "PTB4d" else None
        xcur, _ = _run_ptb(
            xcur, pf[lf], iself[lf], params[ptbn], ds[lf], B * Ns[lf], Ks[lf],
            fin=fin,
        )

    ncls = params["out"]["W"].shape[1]
    return xcur.reshape(B, N, ncls), p
